# baseline (speedup 1.0000x reference)
import functools

import jax
import jax.numpy as jnp
from jax import lax
from jax.experimental import pallas as pl
from jax.experimental.pallas import tpu as pltpu
from jax.experimental.pallas import tpu_sc as plsc

B, H, W = 4, 512, 512
RADIUS = 2
TOP_K = 4096
TEMP = 0.1
KSZ = 2 * RADIUS + 1

HW = H * W
CAP_STRIP = 4096
KCAP = 6144
WCAP = 1536
BAND_ROWS = 70
BAND_ELEMS = BAND_ROWS * W
NBKT = 1024
RES_HALF = 4 * TOP_K + 128
RES_SLICE = 2 * RES_HALF // 16
FLIP = 0x3F800000


def _maxpool5(x):
  neg_r = jnp.full((RADIUS, W), -1.0, jnp.float32)
  xp = jnp.concatenate([neg_r, x, neg_r], axis=0)
  r = xp[0:H]
  for i in range(1, KSZ):
    r = jnp.maximum(r, xp[i:i + H])
  neg_c = jnp.full((H, RADIUS), -1.0, jnp.float32)
  rp = jnp.concatenate([neg_c, r, neg_c], axis=1)
  c = rp[:, 0:W]
  for i in range(1, KSZ):
    c = jnp.maximum(c, rp[:, i:i + W])
  return c


def _nms_body(s_ref, o_ref):
  s = s_ref[0]
  max_mask = s == _maxpool5(s)
  for _ in range(2):
    supp = _maxpool5(max_mask.astype(jnp.float32)) > 0
    ss = jnp.where(supp, 0.0, s)
    new_max = ss == _maxpool5(ss)
    max_mask = max_mask | (new_max & (~supp))
  ri = lax.broadcasted_iota(jnp.int32, (H, W), 0)
  ci = lax.broadcasted_iota(jnp.int32, (H, W), 1)
  border = (ri >= RADIUS) & (ri < H - RADIUS) & (ci >= RADIUS) & (ci < W - RADIUS)
  bits = lax.bitcast_convert_type(s, jnp.int32)
  o_ref[0] = jnp.where(max_mask & border & (s > 0.0), FLIP - bits, FLIP)


def _nms_call(s3):
  return pl.pallas_call(
      _nms_body,
      grid=(B,),
      in_specs=[pl.BlockSpec((1, H, W), lambda b: (b, 0, 0))],
      out_specs=pl.BlockSpec((1, H, W), lambda b: (b, 0, 0)),
      out_shape=jax.ShapeDtypeStruct((B, H, W), jnp.int32),
  )(s3)


def _sc_body(nms_hbm, img_hbm, out_hbm,
             sbuf, ck, ci, hist, run, ka, ia, kb, ib, cntbuf,
             band, tk, wl_pos, wl_idx, patch, st, rid, rchunk,
             spm_t, spm_i, spm_cnt, spm_topk, spm_res, sem):
  c = lax.axis_index("c")
  s = lax.axis_index("s")
  h = s // 8
  strip = s % 8
  img = 2 * c + h
  lane = lax.iota(jnp.int32, 16)

  def zgrp(j, _):
    st[pl.ds(j * 16, 16)] = jnp.zeros((16,), jnp.float32)
    return 0

  lax.fori_loop(0, (RES_SLICE + 15) // 16, zgrp, 0)
  pltpu.sync_copy(st.at[pl.ds(0, RES_SLICE)],
                  spm_res.at[pl.ds(s * RES_SLICE, RES_SLICE)])

  r0s = jnp.minimum(jnp.maximum(64 * strip - 2, 0), H - BAND_ROWS)
  band_cp = pltpu.async_copy(
      img_hbm.at[pl.ds(img * HW + r0s * W, BAND_ELEMS)], band, sem)

  def compact_half(c2, cnt):
    base_off = img * HW + strip * 32768 + c2 * 16384
    pltpu.sync_copy(nms_hbm.at[pl.ds(base_off, 16384)], sbuf)

    def grp(g, cnt):
      t = sbuf[pl.ds(g * 16, 16)]
      m = (t < FLIP) & (cnt < CAP_STRIP - 16)
      pix = strip * 32768 + c2 * 16384 + g * 16 + lane
      occ = plsc.cumsum(jnp.where(m, 1, 0))
      pos = cnt + occ - 1
      plsc.store_scatter(ck, [pos], t, mask=m)
      plsc.store_scatter(ci, [pos], pix, mask=m)
      return cnt + plsc.all_reduce_population_count(m)

    return lax.fori_loop(0, 1024, grp, cnt)

  with jax.named_scope("ph1_compact"):
    cnt = lax.fori_loop(0, 2, compact_half, jnp.zeros((16,), jnp.int32))
  cntbuf[...] = cnt
  pltpu.sync_copy(ck, spm_t.at[h, strip])
  pltpu.sync_copy(ci, spm_i.at[h, strip])
  pltpu.sync_copy(cntbuf, spm_cnt.at[h, strip])

  plsc.subcore_barrier()

  def _sorter_body():
    def zero_hist(ref):
      def z(j, _):
        ref[pl.ds(j * 16, 16)] = jnp.zeros((16,), jnp.int32)
        return 0
      lax.fori_loop(0, NBKT // 16, z, 0)

    zero_hist(hist)

    def hist_strip(stp, _):
      pltpu.sync_copy(spm_cnt.at[h, stp], cntbuf)
      c_st = jnp.minimum(jnp.max(cntbuf[...]), CAP_STRIP)
      pltpu.sync_copy(spm_t.at[h, stp], ck)

      def grp(g, _):
        t = ck[pl.ds(g * 16, 16)]
        m = (g * 16 + lane) < c_st
        bkt = jnp.clip(lax.shift_right_logical(t, 20), 0, NBKT - 1)
        occ, lastm = plsc.scan_count(bkt, m)
        plsc.addupdate_scatter(hist, [bkt], occ, mask=lastm & m)
        return 0

      lax.fori_loop(0, (c_st + 15) // 16, grp, 0)
      return 0

    lax.fori_loop(0, 8, hist_strip, 0)

    def prefix(j, tot):
      hv = hist[pl.ds(j * 16, 16)]
      incl = plsc.cumsum(hv)
      run[pl.ds(j * 16, 16)] = tot + incl - hv
      return tot + jnp.max(incl)

    lax.fori_loop(0, NBKT // 16, prefix, jnp.int32(0))

    def cntc(j, acc):
      exv = run[pl.ds(j * 16, 16)]
      return acc + jnp.max(plsc.all_reduce_population_count(exv < TOP_K))

    c1 = lax.fori_loop(0, NBKT // 16, cntc, jnp.int32(0)) - 1

    def bsum(j, acc):
      hv = hist[pl.ds(j * 16, 16)]
      mv = (j * 16 + lane) < c1
      return acc + jnp.sum(jnp.where(mv, hv, 0))

    base1 = lax.fori_loop(0, NBKT // 16, bsum, jnp.int32(0))

    zero_hist(hist)

    def hist2_strip(stp, _):
      pltpu.sync_copy(spm_cnt.at[h, stp], cntbuf)
      c_st = jnp.minimum(jnp.max(cntbuf[...]), CAP_STRIP)
      pltpu.sync_copy(spm_t.at[h, stp], ck)

      def grp(g, _):
        t = ck[pl.ds(g * 16, 16)]
        m = (g * 16 + lane) < c_st
        bkt = jnp.clip(lax.shift_right_logical(t, 20), 0, NBKT - 1)
        b2 = lax.shift_right_logical(t, 10) & (NBKT - 1)
        m2 = m & (bkt == c1)
        occ, lastm = plsc.scan_count(b2, m2)
        plsc.addupdate_scatter(hist, [b2], occ, mask=lastm & m2)
        return 0

      lax.fori_loop(0, (c_st + 15) // 16, grp, 0)
      return 0

    lax.fori_loop(0, 8, hist2_strip, 0)
    lax.fori_loop(0, NBKT // 16, prefix, base1)

    def keep_strip(stp, kcnt):
      pltpu.sync_copy(spm_cnt.at[h, stp], cntbuf)
      c_st = jnp.minimum(jnp.max(cntbuf[...]), CAP_STRIP)
      pltpu.sync_copy(spm_t.at[h, stp], ck)
      pltpu.sync_copy(spm_i.at[h, stp], ci)

      def grp(g, kcnt):
        t = ck[pl.ds(g * 16, 16)]
        ix = ci[pl.ds(g * 16, 16)]
        m = (g * 16 + lane) < c_st
        bkt = jnp.clip(lax.shift_right_logical(t, 20), 0, NBKT - 1)
        b2 = lax.shift_right_logical(t, 10) & (NBKT - 1)
        ex2 = plsc.load_gather(run, [b2], mask=m)
        keep = m & ((bkt < c1) | ((bkt == c1) & (ex2 < TOP_K))) & (
            kcnt < KCAP - 16)
        occ = plsc.cumsum(jnp.where(keep, 1, 0))
        pos = kcnt + occ - 1
        plsc.store_scatter(ka, [pos], t, mask=keep)
        plsc.store_scatter(ia, [pos], ix, mask=keep)
        return kcnt + plsc.all_reduce_population_count(keep)

      return lax.fori_loop(0, (c_st + 15) // 16, grp, kcnt)

    n_keep = jnp.max(
        lax.fori_loop(0, 8, keep_strip, jnp.zeros((16,), jnp.int32)))
    ngrp = (n_keep + 15) // 16

    def radix_pass(shift, src_k, src_i, dst_k, dst_i):
      zero_hist(hist)

      def hgrp(g, _):
        t = src_k[pl.ds(g * 16, 16)]
        m = (g * 16 + lane) < n_keep
        d = jnp.clip(
            lax.shift_right_logical(t, shift) & (NBKT - 1), 0, NBKT - 1)
        occ, lastm = plsc.scan_count(d, m)
        plsc.addupdate_scatter(hist, [d], occ, mask=lastm & m)
        return 0

      lax.fori_loop(0, ngrp, hgrp, 0)
      lax.fori_loop(0, NBKT // 16, _prefix_into_run, jnp.int32(0))

      def dgrp(g, _):
        t = src_k[pl.ds(g * 16, 16)]
        ix = src_i[pl.ds(g * 16, 16)]
        m = (g * 16 + lane) < n_keep
        d = jnp.clip(
            lax.shift_right_logical(t, shift) & (NBKT - 1), 0, NBKT - 1)
        occ, lastm = plsc.scan_count(d, m)
        cur = plsc.load_gather(run, [d], mask=m)
        pos = jnp.clip(cur + occ - 1, 0, KCAP - 1)
        plsc.store_scatter(dst_k, [pos], t, mask=m)
        plsc.store_scatter(dst_i, [pos], ix, mask=m)
        plsc.addupdate_scatter(run, [d], occ, mask=lastm & m)
        return 0

      lax.fori_loop(0, ngrp, dgrp, 0)

    def _prefix_into_run(j, tot):
      hv = hist[pl.ds(j * 16, 16)]
      incl = plsc.cumsum(hv)
      run[pl.ds(j * 16, 16)] = tot + incl - hv
      return tot + jnp.max(incl)

    radix_pass(0, ka, ia, kb, ib)
    radix_pass(10, kb, ib, ka, ia)

    @pl.when(c1 > 0)
    def _pass3():
      radix_pass(20, ka, ia, kb, ib)

    def publish(src):
      def tkgrp(g, _):
        posv = g * 16 + lane
        ix = src[pl.ds(g * 16, 16)]
        ck[pl.ds(g * 16, 16)] = jnp.where(posv < n_keep, ix, posv - n_keep)
        return 0

      lax.fori_loop(0, TOP_K // 16, tkgrp, 0)

    @pl.when(c1 == 0)
    def _pub2():
      publish(ia)

    @pl.when(c1 > 0)
    def _pub3():
      publish(ib)

    pltpu.sync_copy(ck, spm_topk.at[h])

  @pl.when(strip == 0)
  def _sorter():
    with jax.named_scope("ph2_sort"):
      _sorter_body()

  plsc.subcore_barrier()

  band_cp.wait()
  pltpu.sync_copy(spm_topk.at[h], tk)

  def scan_grp(g, wcnt):
    ix = tk[pl.ds(g * 16, 16)]
    row = lax.shift_right_logical(ix, 9)
    m = (lax.shift_right_logical(row, 6) == strip) & (wcnt < WCAP - 16)
    occ = plsc.cumsum(jnp.where(m, 1, 0))
    pos = wcnt + occ - 1
    plsc.store_scatter(wl_pos, [pos], g * 16 + lane, mask=m)
    plsc.store_scatter(wl_idx, [pos], ix, mask=m)
    return wcnt + plsc.all_reduce_population_count(m)

  with jax.named_scope("ph3_scan"):
    wcnt = jnp.max(
        lax.fori_loop(0, TOP_K // 16, scan_grp, jnp.zeros((16,), jnp.int32)))

  dump = h * RES_HALF + 4 * TOP_K

  def rid_init(j, _):
    rid[pl.ds(j * 16, 16)] = dump + ((j * 16 + lane) & 63)
    return 0

  lax.fori_loop(0, WCAP * 4 // 16, rid_init, 0)

  offs = [(dy, dx) for dy in range(KSZ) for dx in range(KSZ)]

  def refine_grp(g, _):
    i_loc = g * 16 + lane
    m = i_loc < wcnt
    pos = wl_pos[pl.ds(g * 16, 16)]
    ix = wl_idx[pl.ds(g * 16, 16)]
    row = lax.shift_right_logical(ix, 9)
    col = ix & (W - 1)

    mx = jnp.zeros((16,), jnp.float32)
    for t, (dy, dx) in enumerate(offs):
      rr = row + (dy - RADIUS)
      cc = col + (dx - RADIUS)
      inb = m & (rr >= 0) & (rr < H) & (cc >= 0) & (cc < W)
      gidx = jnp.clip((rr - r0s) * W + cc, 0, BAND_ELEMS - 1)
      v = plsc.load_gather(band, [gidx], mask=inb)
      v = jnp.where(inb, v, 0.0)
      patch[t] = v
      mx = jnp.maximum(mx, v)

    s0 = jnp.zeros((16,), jnp.float32)
    sx = jnp.zeros((16,), jnp.float32)
    sy = jnp.zeros((16,), jnp.float32)
    s2 = jnp.zeros((16,), jnp.float32)
    for t, (dy, dx) in enumerate(offs):
      dxf = float(dx - RADIUS)
      dyf = float(dy - RADIUS)
      e = jnp.exp((patch[t] - mx) * (1.0 / TEMP))
      s0 = s0 + e
      sx = sx + e * dxf
      sy = sy + e * dyf
      s2 = s2 + e * (dxf * dxf + dyf * dyf)

    rx = sx / s0
    ry = sy / s0
    disp = (s2 / s0 - rx * rx - ry * ry) * (1.0 / (RADIUS * RADIUS))
    colf = col.astype(jnp.float32)
    rowf = row.astype(jnp.float32)
    kx = (colf + rx) / (W - 1) * 2.0 - 1.0
    ky = (rowf + ry) / (H - 1) * 2.0 - 1.0

    x = (kx + 1.0) * 0.5 * (W - 1)
    y = (ky + 1.0) * 0.5 * (H - 1)
    xt = x.astype(jnp.int32).astype(jnp.float32)
    x0f = jnp.where(x < xt, xt - 1.0, xt)
    yt = y.astype(jnp.int32).astype(jnp.float32)
    y0f = jnp.where(y < yt, yt - 1.0, yt)
    x1f = x0f + 1.0
    y1f = y0f + 1.0
    x0i = jnp.clip(x0f.astype(jnp.int32), 0, W - 1)
    x1i = jnp.clip(x1f.astype(jnp.int32), 0, W - 1)
    y0i = jnp.clip(y0f.astype(jnp.int32), 0, H - 1)
    y1i = jnp.clip(y1f.astype(jnp.int32), 0, H - 1)

    def samp(yi, xi):
      gi = jnp.clip((yi - r0s) * W + xi, 0, BAND_ELEMS - 1)
      v = plsc.load_gather(band, [gi], mask=m)
      return jnp.where(m, v, 0.0)

    wa = (x1f - x) * (y1f - y)
    wb = (x1f - x) * (y - y0f)
    wc = (x - x0f) * (y1f - y)
    wd = (x - x0f) * (y - y0f)
    ks = (wa * samp(y0i, x0i) + wb * samp(y1i, x0i)
          + wc * samp(y0i, x1i) + wd * samp(y1i, x1i))

    i4 = i_loc * 4
    p4 = h * RES_HALF + pos * 4
    for q, val in enumerate((kx, ky, ks, disp)):
      f = i4 + q
      plsc.store_scatter(st, [f], val, mask=m)
      plsc.store_scatter(rid, [f], p4 + q, mask=m)
    return 0

  with jax.named_scope("ph3_refine"):
    lax.fori_loop(0, (wcnt + 15) // 16, refine_grp, 0)

  for k in range(WCAP * 4 // 512):
    @pl.when(wcnt * 4 > k * 512)
    def _scat(k=k):
      def cp(j, _):
        rchunk[pl.ds(j * 16, 16)] = rid[pl.ds(k * 512 + j * 16, 16)]
        return 0

      lax.fori_loop(0, 32, cp, 0)
      pltpu.sync_copy(st.at[pl.ds(k * 512, 512)], spm_res.at[rchunk],
                      add=True)

  plsc.subcore_barrier()

  @pl.when(strip == 0)
  def _writeback():
    pltpu.sync_copy(spm_res.at[pl.ds(h * RES_HALF, 4 * TOP_K)], out_hbm.at[img])


@functools.partial(
    pl.kernel,
    out_type=jax.ShapeDtypeStruct((B, 4 * TOP_K), jnp.float32),
    mesh=plsc.VectorSubcoreMesh(core_axis_name="c", subcore_axis_name="s"),
    compiler_params=pltpu.CompilerParams(needs_layout_passes=False),
    scratch_types=[
        pltpu.VMEM((16384,), jnp.int32),
        pltpu.VMEM((CAP_STRIP,), jnp.int32),
        pltpu.VMEM((CAP_STRIP,), jnp.int32),
        pltpu.VMEM((NBKT,), jnp.int32),
        pltpu.VMEM((NBKT,), jnp.int32),
        pltpu.VMEM((KCAP,), jnp.int32),
        pltpu.VMEM((KCAP,), jnp.int32),
        pltpu.VMEM((KCAP,), jnp.int32),
        pltpu.VMEM((KCAP,), jnp.int32),
        pltpu.VMEM((16,), jnp.int32),
        pltpu.VMEM((BAND_ELEMS,), jnp.float32),
        pltpu.VMEM((TOP_K,), jnp.int32),
        pltpu.VMEM((WCAP,), jnp.int32),
        pltpu.VMEM((WCAP,), jnp.int32),
        pltpu.VMEM((25, 16), jnp.float32),
        pltpu.VMEM((WCAP * 4,), jnp.float32),
        pltpu.VMEM((WCAP * 4,), jnp.int32),
        pltpu.VMEM((512,), jnp.int32),
        pltpu.VMEM_SHARED((2, 8, CAP_STRIP), jnp.int32),
        pltpu.VMEM_SHARED((2, 8, CAP_STRIP), jnp.int32),
        pltpu.VMEM_SHARED((2, 8, 16), jnp.int32),
        pltpu.VMEM_SHARED((2, TOP_K), jnp.int32),
        pltpu.VMEM_SHARED((2 * RES_HALF,), jnp.float32),
        pltpu.SemaphoreType.DMA,
    ],
)
def _sc_call(nms_hbm, img_hbm, out_hbm, *scratch):
  _sc_body(nms_hbm, img_hbm, out_hbm, *scratch)


@jax.jit
def kernel(scores_map):
  s3 = scores_map.reshape(B, H, W)
  nms = _nms_call(s3)
  out = _sc_call(nms.reshape(B * HW), s3.reshape(B * HW))
  o = out.reshape(B, TOP_K, 4)
  return o[:, :, 0:2], o[:, :, 2], o[:, :, 3]


# speedup vs baseline: 1.2608x; 1.2608x over previous
import functools

import jax
import jax.numpy as jnp
from jax import lax
from jax.experimental import pallas as pl
from jax.experimental.pallas import tpu as pltpu
from jax.experimental.pallas import tpu_sc as plsc

B, H, W = 4, 512, 512
RADIUS = 2
TOP_K = 4096
TEMP = 0.1
KSZ = 2 * RADIUS + 1

HW = H * W
CAP_STRIP = 4096
KEEP_CAP = 1024
KCAP = 8192
WCAP = 1536
BAND_ROWS = 70
BAND_ELEMS = BAND_ROWS * W
NBKT = 1024
RES_HALF = 4 * TOP_K + 128
RES_SLICE = 2 * RES_HALF // 16
FLIP = 0x3F800000
SENT = 0x3FFFFFFF


def _maxpool5(x):
  neg_r = jnp.full((RADIUS, W), -1.0, jnp.float32)
  xp = jnp.concatenate([neg_r, x, neg_r], axis=0)
  r = xp[0:H]
  for i in range(1, KSZ):
    r = jnp.maximum(r, xp[i:i + H])
  neg_c = jnp.full((H, RADIUS), -1.0, jnp.float32)
  rp = jnp.concatenate([neg_c, r, neg_c], axis=1)
  c = rp[:, 0:W]
  for i in range(1, KSZ):
    c = jnp.maximum(c, rp[:, i:i + W])
  return c


def _nms_body(s_ref, o_ref):
  s = s_ref[0]
  max_mask = s == _maxpool5(s)
  for _ in range(2):
    supp = _maxpool5(max_mask.astype(jnp.float32)) > 0
    ss = jnp.where(supp, 0.0, s)
    new_max = ss == _maxpool5(ss)
    max_mask = max_mask | (new_max & (~supp))
  ri = lax.broadcasted_iota(jnp.int32, (H, W), 0)
  ci = lax.broadcasted_iota(jnp.int32, (H, W), 1)
  border = (ri >= RADIUS) & (ri < H - RADIUS) & (ci >= RADIUS) & (ci < W - RADIUS)
  bits = lax.bitcast_convert_type(s, jnp.int32)
  o_ref[0] = jnp.where(max_mask & border & (s > 0.0), FLIP - bits, FLIP)


def _nms_call(s3):
  return pl.pallas_call(
      _nms_body,
      grid=(B,),
      in_specs=[pl.BlockSpec((1, H, W), lambda b: (b, 0, 0))],
      out_specs=pl.BlockSpec((1, H, W), lambda b: (b, 0, 0)),
      out_shape=jax.ShapeDtypeStruct((B, H, W), jnp.int32),
  )(s3)


def _sc_body(nms_hbm, img_hbm, out_hbm,
             sbuf, ck, ci, hist, run, ka, ia, kb, ib, cntbuf,
             band, tk, wl_pos, wl_idx, patch, st, rid, rchunk,
             spm_kt, spm_ki, spm_cnt, spm_h1, spm_topk, spm_res, sem):
  c = lax.axis_index("c")
  s = lax.axis_index("s")
  h = s // 8
  strip = s % 8
  img = 2 * c + h
  lane = lax.iota(jnp.int32, 16)

  def zgrp(j, _):
    st[pl.ds(j * 16, 16)] = jnp.zeros((16,), jnp.float32)
    return 0

  lax.fori_loop(0, (RES_SLICE + 15) // 16, zgrp, 0)
  pltpu.sync_copy(st.at[pl.ds(0, RES_SLICE)],
                  spm_res.at[pl.ds(s * RES_SLICE, RES_SLICE)])

  r0s = jnp.minimum(jnp.maximum(64 * strip - 2, 0), H - BAND_ROWS)
  band_cp = pltpu.async_copy(
      img_hbm.at[pl.ds(img * HW + r0s * W, BAND_ELEMS)], band, sem)

  def zero_hist(ref):
    def z(j, _):
      ref[pl.ds(j * 16, 16)] = jnp.zeros((16,), jnp.int32)
      return 0
    lax.fori_loop(0, NBKT // 16, z, 0)

  def prefix_into_run(j, tot):
    hv = hist[pl.ds(j * 16, 16)]
    incl = plsc.cumsum(hv)
    run[pl.ds(j * 16, 16)] = tot + incl - hv
    return tot + jnp.max(incl)

  def merge_h1():
    def mj(j, _):
      acc = sbuf[pl.ds(2048 + j * 16, 16)]
      for stp in range(1, 8):
        acc = acc + sbuf[pl.ds(2048 + stp * NBKT + j * 16, 16)]
      hist[pl.ds(j * 16, 16)] = acc
      return 0
    lax.fori_loop(0, NBKT // 16, mj, 0)

  def compact_half(c2, cnt):
    base_off = img * HW + strip * 32768 + c2 * 16384
    pltpu.sync_copy(nms_hbm.at[pl.ds(base_off, 16384)], sbuf)

    def grp(g, cnt):
      t = sbuf[pl.ds(g * 16, 16)]
      m = (t < FLIP) & (cnt < CAP_STRIP - 16)
      pix = strip * 32768 + c2 * 16384 + g * 16 + lane
      occ = plsc.cumsum(jnp.where(m, 1, 0))
      pos = cnt + occ - 1
      plsc.store_scatter(ck, [pos], t, mask=m)
      plsc.store_scatter(ci, [pos], pix, mask=m)
      return cnt + plsc.all_reduce_population_count(m)

    return lax.fori_loop(0, 1024, grp, cnt)

  with jax.named_scope("ph1_compact"):
    cnt = lax.fori_loop(0, 2, compact_half, jnp.zeros((16,), jnp.int32))
  cs = jnp.max(cnt)
  csg = (cs + 15) // 16

  with jax.named_scope("ph1_hist"):
    zero_hist(hist)

    def h1grp(g, _):
      t = ck[pl.ds(g * 16, 16)]
      m = (g * 16 + lane) < cs
      bkt = jnp.clip(lax.shift_right_logical(t, 20), 0, NBKT - 1)
      occ, lastm = plsc.scan_count(bkt, m)
      plsc.addupdate_scatter(hist, [bkt], occ, mask=lastm & m)
      return 0

    lax.fori_loop(0, csg, h1grp, 0)
  pltpu.sync_copy(hist, spm_h1.at[h, pl.ds(strip * NBKT, NBKT)])

  plsc.subcore_barrier()

  with jax.named_scope("ph15_merge"):
    pltpu.sync_copy(spm_h1.at[h], sbuf.at[pl.ds(2048, 8 * NBKT)])
    merge_h1()
    lax.fori_loop(0, NBKT // 16, prefix_into_run, jnp.int32(0))

    def cntc(j, acc):
      exv = run[pl.ds(j * 16, 16)]
      return acc + plsc.all_reduce_population_count(exv < TOP_K)

    c1 = jnp.max(
        lax.fori_loop(0, NBKT // 16, cntc, jnp.zeros((16,), jnp.int32))) - 1

    def bsum(j, acc):
      hv = hist[pl.ds(j * 16, 16)]
      mv = (j * 16 + lane) < c1
      return acc + jnp.sum(jnp.where(mv, hv, 0))

    base1 = lax.fori_loop(0, NBKT // 16, bsum, jnp.int32(0))

  plsc.subcore_barrier()

  with jax.named_scope("ph15_hist2"):
    zero_hist(hist)

    def h2grp(g, _):
      t = ck[pl.ds(g * 16, 16)]
      m = (g * 16 + lane) < cs
      bkt = jnp.clip(lax.shift_right_logical(t, 20), 0, NBKT - 1)
      b2 = lax.shift_right_logical(t, 10) & (NBKT - 1)
      m2 = m & (bkt == c1)
      occ, lastm = plsc.scan_count(b2, m2)
      plsc.addupdate_scatter(hist, [b2], occ, mask=lastm & m2)
      return 0

    lax.fori_loop(0, csg, h2grp, 0)
  pltpu.sync_copy(hist, spm_h1.at[h, pl.ds(strip * NBKT, NBKT)])

  plsc.subcore_barrier()

  with jax.named_scope("ph16_keep"):
    pltpu.sync_copy(spm_h1.at[h], sbuf.at[pl.ds(2048, 8 * NBKT)])
    merge_h1()
    lax.fori_loop(0, NBKT // 16, prefix_into_run, base1)

    def sfill(j, _):
      sbuf[pl.ds(j * 16, 16)] = jnp.zeros((16,), jnp.int32) + SENT
      sbuf[pl.ds(KEEP_CAP + j * 16, 16)] = jnp.zeros((16,), jnp.int32)
      return 0

    lax.fori_loop(0, KEEP_CAP // 16, sfill, 0)

    def kgrp(g, kcnt):
      t = ck[pl.ds(g * 16, 16)]
      ix = ci[pl.ds(g * 16, 16)]
      m = (g * 16 + lane) < cs
      bkt = jnp.clip(lax.shift_right_logical(t, 20), 0, NBKT - 1)
      b2 = lax.shift_right_logical(t, 10) & (NBKT - 1)
      ex2 = plsc.load_gather(run, [b2], mask=m)
      keep = m & ((bkt < c1) | ((bkt == c1) & (ex2 < TOP_K))) & (
          kcnt < KEEP_CAP - 16)
      occ = plsc.cumsum(jnp.where(keep, 1, 0))
      pos = kcnt + occ - 1
      plsc.store_scatter(sbuf, [pos], t, mask=keep)
      plsc.store_scatter(sbuf, [KEEP_CAP + pos], ix, mask=keep)
      return kcnt + plsc.all_reduce_population_count(keep)

    kcnt = lax.fori_loop(0, csg, kgrp, jnp.zeros((16,), jnp.int32))
    cntbuf[...] = kcnt
  pltpu.sync_copy(sbuf.at[pl.ds(0, KEEP_CAP)], spm_kt.at[h, strip])
  pltpu.sync_copy(sbuf.at[pl.ds(KEEP_CAP, KEEP_CAP)], spm_ki.at[h, strip])
  pltpu.sync_copy(cntbuf, spm_cnt.at[h, pl.ds(strip * 16, 16)])

  plsc.subcore_barrier()

  def _sorter_body():
    pltpu.sync_copy(spm_cnt.at[h], sbuf.at[pl.ds(10240, 128)])
    off = jnp.int32(0)
    n_real = jnp.int32(0)
    n_eff = jnp.int32(0)
    for stp in range(8):
      kc = jnp.minimum(jnp.max(sbuf[pl.ds(10240 + stp * 16, 16)]), KEEP_CAP)
      pltpu.sync_copy(spm_kt.at[h, stp], ka.at[pl.ds(off, KEEP_CAP)])
      pltpu.sync_copy(spm_ki.at[h, stp], ia.at[pl.ds(off, KEEP_CAP)])
      if stp == 7:
        n_eff = off + KEEP_CAP
      off = off + ((kc + 127) // 128) * 128
      n_real = n_real + kc
    ngrp = (n_eff + 15) // 16

    def radix_pass(shift, src_k, src_i, dst_k, dst_i):
      zero_hist(hist)

      def hgrp(g, _):
        t = src_k[pl.ds(g * 16, 16)]
        m = ((g * 16 + lane) < n_eff) & (t != SENT)
        d = jnp.clip(
            lax.shift_right_logical(t, shift) & (NBKT - 1), 0, NBKT - 1)
        occ, lastm = plsc.scan_count(d, m)
        plsc.addupdate_scatter(hist, [d], occ, mask=lastm & m)
        return 0

      lax.fori_loop(0, ngrp, hgrp, 0)
      lax.fori_loop(0, NBKT // 16, prefix_into_run, jnp.int32(0))

      def dgrp(g, _):
        t = src_k[pl.ds(g * 16, 16)]
        ix = src_i[pl.ds(g * 16, 16)]
        m = ((g * 16 + lane) < n_eff) & (t != SENT)
        d = jnp.clip(
            lax.shift_right_logical(t, shift) & (NBKT - 1), 0, NBKT - 1)
        occ, lastm = plsc.scan_count(d, m)
        cur = plsc.load_gather(run, [d], mask=m)
        pos = jnp.clip(cur + occ - 1, 0, KCAP - 1)
        plsc.store_scatter(dst_k, [pos], t, mask=m)
        plsc.store_scatter(dst_i, [pos], ix, mask=m)
        plsc.addupdate_scatter(run, [d], occ, mask=lastm & m)
        return 0

      lax.fori_loop(0, ngrp, dgrp, 0)

    radix_pass(0, ka, ia, kb, ib)
    radix_pass(10, kb, ib, ka, ia)

    @pl.when(c1 > 0)
    def _pass3():
      radix_pass(20, ka, ia, kb, ib)

    def publish(src):
      def tkgrp(g, _):
        posv = g * 16 + lane
        ix = src[pl.ds(g * 16, 16)]
        ck[pl.ds(g * 16, 16)] = jnp.where(posv < n_real, ix, posv - n_real)
        return 0

      lax.fori_loop(0, TOP_K // 16, tkgrp, 0)

    @pl.when(c1 == 0)
    def _pub2():
      publish(ia)

    @pl.when(c1 > 0)
    def _pub3():
      publish(ib)

    pltpu.sync_copy(ck, spm_topk.at[h])

  @pl.when(strip == 0)
  def _sorter():
    with jax.named_scope("ph2_sort"):
      _sorter_body()

  plsc.subcore_barrier()

  band_cp.wait()
  pltpu.sync_copy(spm_topk.at[h], tk)

  def scan_grp(g, wcnt):
    ix = tk[pl.ds(g * 16, 16)]
    row = lax.shift_right_logical(ix, 9)
    m = (lax.shift_right_logical(row, 6) == strip) & (wcnt < WCAP - 16)
    occ = plsc.cumsum(jnp.where(m, 1, 0))
    pos = wcnt + occ - 1
    plsc.store_scatter(wl_pos, [pos], g * 16 + lane, mask=m)
    plsc.store_scatter(wl_idx, [pos], ix, mask=m)
    return wcnt + plsc.all_reduce_population_count(m)

  with jax.named_scope("ph3_scan"):
    wcnt = jnp.max(
        lax.fori_loop(0, TOP_K // 16, scan_grp, jnp.zeros((16,), jnp.int32)))

  dump = h * RES_HALF + 4 * TOP_K

  def rid_init(j, _):
    rid[pl.ds(j * 16, 16)] = dump + ((j * 16 + lane) & 63)
    return 0

  lax.fori_loop(0, WCAP * 4 // 16, rid_init, 0)

  offs = [(dy, dx) for dy in range(KSZ) for dx in range(KSZ)]

  def refine_grp(g, _):
    i_loc = g * 16 + lane
    m = i_loc < wcnt
    pos = wl_pos[pl.ds(g * 16, 16)]
    ix = wl_idx[pl.ds(g * 16, 16)]
    row = lax.shift_right_logical(ix, 9)
    col = ix & (W - 1)

    mx = jnp.zeros((16,), jnp.float32)
    for t, (dy, dx) in enumerate(offs):
      rr = row + (dy - RADIUS)
      cc = col + (dx - RADIUS)
      inb = m & (rr >= 0) & (rr < H) & (cc >= 0) & (cc < W)
      gidx = jnp.clip((rr - r0s) * W + cc, 0, BAND_ELEMS - 1)
      v = plsc.load_gather(band, [gidx], mask=inb)
      v = jnp.where(inb, v, 0.0)
      patch[t] = v
      mx = jnp.maximum(mx, v)

    s0 = jnp.zeros((16,), jnp.float32)
    sx = jnp.zeros((16,), jnp.float32)
    sy = jnp.zeros((16,), jnp.float32)
    s2 = jnp.zeros((16,), jnp.float32)
    for t, (dy, dx) in enumerate(offs):
      dxf = float(dx - RADIUS)
      dyf = float(dy - RADIUS)
      e = jnp.exp((patch[t] - mx) * (1.0 / TEMP))
      s0 = s0 + e
      sx = sx + e * dxf
      sy = sy + e * dyf
      s2 = s2 + e * (dxf * dxf + dyf * dyf)

    rx = sx / s0
    ry = sy / s0
    disp = (s2 / s0 - rx * rx - ry * ry) * (1.0 / (RADIUS * RADIUS))
    colf = col.astype(jnp.float32)
    rowf = row.astype(jnp.float32)
    kx = (colf + rx) / (W - 1) * 2.0 - 1.0
    ky = (rowf + ry) / (H - 1) * 2.0 - 1.0

    x = (kx + 1.0) * 0.5 * (W - 1)
    y = (ky + 1.0) * 0.5 * (H - 1)
    xt = x.astype(jnp.int32).astype(jnp.float32)
    x0f = jnp.where(x < xt, xt - 1.0, xt)
    yt = y.astype(jnp.int32).astype(jnp.float32)
    y0f = jnp.where(y < yt, yt - 1.0, yt)
    x1f = x0f + 1.0
    y1f = y0f + 1.0
    x0i = jnp.clip(x0f.astype(jnp.int32), 0, W - 1)
    x1i = jnp.clip(x1f.astype(jnp.int32), 0, W - 1)
    y0i = jnp.clip(y0f.astype(jnp.int32), 0, H - 1)
    y1i = jnp.clip(y1f.astype(jnp.int32), 0, H - 1)

    def samp(yi, xi):
      gi = jnp.clip((yi - r0s) * W + xi, 0, BAND_ELEMS - 1)
      v = plsc.load_gather(band, [gi], mask=m)
      return jnp.where(m, v, 0.0)

    wa = (x1f - x) * (y1f - y)
    wb = (x1f - x) * (y - y0f)
    wc = (x - x0f) * (y1f - y)
    wd = (x - x0f) * (y - y0f)
    ks = (wa * samp(y0i, x0i) + wb * samp(y1i, x0i)
          + wc * samp(y0i, x1i) + wd * samp(y1i, x1i))

    i4 = i_loc * 4
    p4 = h * RES_HALF + pos * 4
    for q, val in enumerate((kx, ky, ks, disp)):
      f = i4 + q
      plsc.store_scatter(st, [f], val, mask=m)
      plsc.store_scatter(rid, [f], p4 + q, mask=m)
    return 0

  with jax.named_scope("ph3_refine"):
    lax.fori_loop(0, (wcnt + 15) // 16, refine_grp, 0)

  with jax.named_scope("ph3_scatter"):
    for k in range(WCAP * 4 // 512):
      @pl.when(wcnt * 4 > k * 512)
      def _scat(k=k):
        def cp(j, _):
          rchunk[pl.ds(j * 16, 16)] = rid[pl.ds(k * 512 + j * 16, 16)]
          return 0

        lax.fori_loop(0, 32, cp, 0)
        pltpu.sync_copy(st.at[pl.ds(k * 512, 512)], spm_res.at[rchunk],
                        add=True)

  plsc.subcore_barrier()

  @pl.when(strip == 0)
  def _writeback():
    pltpu.sync_copy(spm_res.at[pl.ds(h * RES_HALF, 4 * TOP_K)], out_hbm.at[img])


@functools.partial(
    pl.kernel,
    out_type=jax.ShapeDtypeStruct((B, 4 * TOP_K), jnp.float32),
    mesh=plsc.VectorSubcoreMesh(core_axis_name="c", subcore_axis_name="s"),
    compiler_params=pltpu.CompilerParams(needs_layout_passes=False),
    scratch_types=[
        pltpu.VMEM((16384,), jnp.int32),
        pltpu.VMEM((CAP_STRIP,), jnp.int32),
        pltpu.VMEM((CAP_STRIP,), jnp.int32),
        pltpu.VMEM((NBKT,), jnp.int32),
        pltpu.VMEM((NBKT,), jnp.int32),
        pltpu.VMEM((KCAP,), jnp.int32),
        pltpu.VMEM((KCAP,), jnp.int32),
        pltpu.VMEM((KCAP,), jnp.int32),
        pltpu.VMEM((KCAP,), jnp.int32),
        pltpu.VMEM((16,), jnp.int32),
        pltpu.VMEM((BAND_ELEMS,), jnp.float32),
        pltpu.VMEM((TOP_K,), jnp.int32),
        pltpu.VMEM((WCAP,), jnp.int32),
        pltpu.VMEM((WCAP,), jnp.int32),
        pltpu.VMEM((25, 16), jnp.float32),
        pltpu.VMEM((WCAP * 4,), jnp.float32),
        pltpu.VMEM((WCAP * 4,), jnp.int32),
        pltpu.VMEM((512,), jnp.int32),
        pltpu.VMEM_SHARED((2, 8, KEEP_CAP), jnp.int32),
        pltpu.VMEM_SHARED((2, 8, KEEP_CAP), jnp.int32),
        pltpu.VMEM_SHARED((2, 128), jnp.int32),
        pltpu.VMEM_SHARED((2, 8 * NBKT), jnp.int32),
        pltpu.VMEM_SHARED((2, TOP_K), jnp.int32),
        pltpu.VMEM_SHARED((2 * RES_HALF,), jnp.float32),
        pltpu.SemaphoreType.DMA,
    ],
)
def _sc_call(nms_hbm, img_hbm, out_hbm, *scratch):
  _sc_body(nms_hbm, img_hbm, out_hbm, *scratch)


@jax.jit
def kernel(scores_map):
  s3 = scores_map.reshape(B, H, W)
  nms = _nms_call(s3)
  out = _sc_call(nms.reshape(B * HW), s3.reshape(B * HW))
  o = out.reshape(B, TOP_K, 4)
  return o[:, :, 0:2], o[:, :, 2], o[:, :, 3]


# speedup vs baseline: 1.3301x; 1.0549x over previous
import functools

import jax
import jax.numpy as jnp
from jax import lax
from jax.experimental import pallas as pl
from jax.experimental.pallas import tpu as pltpu
from jax.experimental.pallas import tpu_sc as plsc

B, H, W = 4, 512, 512
RADIUS = 2
TOP_K = 4096
TEMP = 0.1
KSZ = 2 * RADIUS + 1

HW = H * W
CAP_STRIP = 4096
KEEP_CAP = 1024
KCAP = 8192
WCAP = 1536
BAND_ROWS = 70
BAND_ELEMS = BAND_ROWS * W
NBKT = 1024
RES_HALF = 4 * TOP_K + 128
RES_SLICE = 2 * RES_HALF // 16
FLIP = 0x3F800000
SENT = 0x3FFFFFFF


def _maxpool5(x):
  neg_r = jnp.full((RADIUS, W), -1.0, jnp.float32)
  xp = jnp.concatenate([neg_r, x, neg_r], axis=0)
  r = xp[0:H]
  for i in range(1, KSZ):
    r = jnp.maximum(r, xp[i:i + H])
  neg_c = jnp.full((H, RADIUS), -1.0, jnp.float32)
  rp = jnp.concatenate([neg_c, r, neg_c], axis=1)
  c = rp[:, 0:W]
  for i in range(1, KSZ):
    c = jnp.maximum(c, rp[:, i:i + W])
  return c


def _nms_body(s_ref, o_ref):
  s = s_ref[0]
  max_mask = s == _maxpool5(s)
  for _ in range(2):
    supp = _maxpool5(max_mask.astype(jnp.float32)) > 0
    ss = jnp.where(supp, 0.0, s)
    new_max = ss == _maxpool5(ss)
    max_mask = max_mask | (new_max & (~supp))
  ri = lax.broadcasted_iota(jnp.int32, (H, W), 0)
  ci = lax.broadcasted_iota(jnp.int32, (H, W), 1)
  border = (ri >= RADIUS) & (ri < H - RADIUS) & (ci >= RADIUS) & (ci < W - RADIUS)
  bits = lax.bitcast_convert_type(s, jnp.int32)
  o_ref[0] = jnp.where(max_mask & border & (s > 0.0), FLIP - bits, FLIP)


def _nms_call(s3):
  return pl.pallas_call(
      _nms_body,
      grid=(B,),
      in_specs=[pl.BlockSpec((1, H, W), lambda b: (b, 0, 0))],
      out_specs=pl.BlockSpec((1, H, W), lambda b: (b, 0, 0)),
      out_shape=jax.ShapeDtypeStruct((B, H, W), jnp.int32),
  )(s3)


def _sc_body(nms_hbm, img_hbm, out_hbm,
             sbuf, ck, ci, hist, run, ka, ia, kb, ib, cntbuf,
             band, tk, wl_pos, wl_idx, patch, st, rid, rchunk,
             spm_kt, spm_ki, spm_cnt, spm_h1, spm_topk, spm_res, sem):
  c = lax.axis_index("c")
  s = lax.axis_index("s")
  h = s // 8
  strip = s % 8
  img = 2 * c + h
  lane = lax.iota(jnp.int32, 16)

  def zgrp(j, _):
    st[pl.ds(j * 16, 16)] = jnp.zeros((16,), jnp.float32)
    return 0

  lax.fori_loop(0, (RES_SLICE + 15) // 16, zgrp, 0)
  pltpu.sync_copy(st.at[pl.ds(0, RES_SLICE)],
                  spm_res.at[pl.ds(s * RES_SLICE, RES_SLICE)])

  r0s = jnp.minimum(jnp.maximum(64 * strip - 2, 0), H - BAND_ROWS)
  band_cp = pltpu.async_copy(
      img_hbm.at[pl.ds(img * HW + r0s * W, BAND_ELEMS)], band, sem)

  def zero_hist(ref):
    def z(j, _):
      ref[pl.ds(j * 16, 16)] = jnp.zeros((16,), jnp.int32)
      return 0
    lax.fori_loop(0, NBKT // 16, z, 0)

  def prefix_into_run(j, tot):
    hv = hist[pl.ds(j * 16, 16)]
    incl = plsc.cumsum(hv)
    run[pl.ds(j * 16, 16)] = tot + incl - hv
    return tot + jnp.max(incl)

  def merge_h1():
    def mj(j, _):
      acc = sbuf[pl.ds(2048 + j * 16, 16)]
      for stp in range(1, 8):
        acc = acc + sbuf[pl.ds(2048 + stp * NBKT + j * 16, 16)]
      hist[pl.ds(j * 16, 16)] = acc
      return 0
    lax.fori_loop(0, NBKT // 16, mj, 0)

  def compact_half(c2, cnt):
    base_off = img * HW + strip * 32768 + c2 * 16384
    pltpu.sync_copy(nms_hbm.at[pl.ds(base_off, 16384)], sbuf)

    def grp(g, cnt):
      t = sbuf[pl.ds(g * 16, 16)]
      m = (t < FLIP) & (cnt < CAP_STRIP - 16)
      pix = strip * 32768 + c2 * 16384 + g * 16 + lane
      occ = plsc.cumsum(jnp.where(m, 1, 0))
      pos = cnt + occ - 1
      plsc.store_scatter(ck, [pos], t, mask=m)
      plsc.store_scatter(ci, [pos], pix, mask=m)
      return cnt + plsc.all_reduce_population_count(m)

    return lax.fori_loop(0, 1024, grp, cnt)

  with jax.named_scope("ph1_compact"):
    cnt = lax.fori_loop(0, 2, compact_half, jnp.zeros((16,), jnp.int32))
  cs = jnp.max(cnt)
  csg = (cs + 15) // 16

  with jax.named_scope("ph1_hist"):
    zero_hist(hist)

    def h1grp(g, _):
      t = ck[pl.ds(g * 16, 16)]
      m = (g * 16 + lane) < cs
      bkt = jnp.clip(lax.shift_right_logical(t, 20), 0, NBKT - 1)
      occ, lastm = plsc.scan_count(bkt, m)
      plsc.addupdate_scatter(hist, [bkt], occ, mask=lastm & m)
      return 0

    lax.fori_loop(0, csg, h1grp, 0)
  pltpu.sync_copy(hist, spm_h1.at[h, pl.ds(strip * NBKT, NBKT)])

  plsc.subcore_barrier()

  with jax.named_scope("ph15_merge"):
    pltpu.sync_copy(spm_h1.at[h], sbuf.at[pl.ds(2048, 8 * NBKT)])
    merge_h1()
    lax.fori_loop(0, NBKT // 16, prefix_into_run, jnp.int32(0))

    def cntc(j, acc):
      exv = run[pl.ds(j * 16, 16)]
      return acc + plsc.all_reduce_population_count(exv < TOP_K)

    c1 = jnp.max(
        lax.fori_loop(0, NBKT // 16, cntc, jnp.zeros((16,), jnp.int32))) - 1

    def bsum(j, acc):
      hv = hist[pl.ds(j * 16, 16)]
      mv = (j * 16 + lane) < c1
      return acc + jnp.sum(jnp.where(mv, hv, 0))

    base1 = lax.fori_loop(0, NBKT // 16, bsum, jnp.int32(0))

  plsc.subcore_barrier()

  with jax.named_scope("ph15_hist2"):
    zero_hist(hist)

    def h2grp(g, _):
      t = ck[pl.ds(g * 16, 16)]
      m = (g * 16 + lane) < cs
      bkt = jnp.clip(lax.shift_right_logical(t, 20), 0, NBKT - 1)
      b2 = lax.shift_right_logical(t, 10) & (NBKT - 1)
      m2 = m & (bkt == c1)
      occ, lastm = plsc.scan_count(b2, m2)
      plsc.addupdate_scatter(hist, [b2], occ, mask=lastm & m2)
      return 0

    lax.fori_loop(0, csg, h2grp, 0)
  pltpu.sync_copy(hist, spm_h1.at[h, pl.ds(strip * NBKT, NBKT)])

  plsc.subcore_barrier()

  with jax.named_scope("ph16_keep"):
    pltpu.sync_copy(spm_h1.at[h], sbuf.at[pl.ds(2048, 8 * NBKT)])
    merge_h1()
    lax.fori_loop(0, NBKT // 16, prefix_into_run, base1)

    def sfill(j, _):
      sbuf[pl.ds(j * 16, 16)] = jnp.zeros((16,), jnp.int32) + SENT
      sbuf[pl.ds(KEEP_CAP + j * 16, 16)] = jnp.zeros((16,), jnp.int32)
      return 0

    lax.fori_loop(0, KEEP_CAP // 16, sfill, 0)

    def kgrp(g, kcnt):
      t = ck[pl.ds(g * 16, 16)]
      ix = ci[pl.ds(g * 16, 16)]
      m = (g * 16 + lane) < cs
      bkt = jnp.clip(lax.shift_right_logical(t, 20), 0, NBKT - 1)
      b2 = lax.shift_right_logical(t, 10) & (NBKT - 1)
      ex2 = plsc.load_gather(run, [b2], mask=m)
      keep = m & ((bkt < c1) | ((bkt == c1) & (ex2 < TOP_K))) & (
          kcnt < KEEP_CAP - 16)
      occ = plsc.cumsum(jnp.where(keep, 1, 0))
      pos = kcnt + occ - 1
      plsc.store_scatter(sbuf, [pos], t, mask=keep)
      plsc.store_scatter(sbuf, [KEEP_CAP + pos], ix, mask=keep)
      return kcnt + plsc.all_reduce_population_count(keep)

    kcnt = lax.fori_loop(0, csg, kgrp, jnp.zeros((16,), jnp.int32))
    cntbuf[...] = kcnt
  pltpu.sync_copy(sbuf.at[pl.ds(0, KEEP_CAP)], spm_kt.at[h, strip])
  pltpu.sync_copy(sbuf.at[pl.ds(KEEP_CAP, KEEP_CAP)], spm_ki.at[h, strip])
  pltpu.sync_copy(cntbuf, spm_cnt.at[h, pl.ds(strip * 16, 16)])

  plsc.subcore_barrier()

  def _sorter_body():
    pltpu.sync_copy(spm_cnt.at[h], sbuf.at[pl.ds(10240, 128)])
    off = jnp.int32(0)
    n_real = jnp.int32(0)
    n_eff = jnp.int32(0)
    for stp in range(8):
      kc = jnp.minimum(jnp.max(sbuf[pl.ds(10240 + stp * 16, 16)]), KEEP_CAP)
      pltpu.sync_copy(spm_kt.at[h, stp], ka.at[pl.ds(off, KEEP_CAP)])
      pltpu.sync_copy(spm_ki.at[h, stp], ia.at[pl.ds(off, KEEP_CAP)])
      if stp == 7:
        n_eff = off + KEEP_CAP
      off = off + ((kc + 127) // 128) * 128
      n_real = n_real + kc
    ngrp = (n_eff + 15) // 16

    def radix_pass(shift, src_k, src_i, dst_k, dst_i, limit, ng, check_sent):
      zero_hist(hist)

      def hgrp(g, _):
        t = src_k[pl.ds(g * 16, 16)]
        m = (g * 16 + lane) < limit
        if check_sent:
          m = m & (t != SENT)
        d = jnp.clip(
            lax.shift_right_logical(t, shift) & (NBKT - 1), 0, NBKT - 1)
        occ, lastm = plsc.scan_count(d, m)
        plsc.addupdate_scatter(hist, [d], occ, mask=lastm & m)
        return 0

      lax.fori_loop(0, ng, hgrp, 0)
      lax.fori_loop(0, NBKT // 16, prefix_into_run, jnp.int32(0))

      def dgrp(g, _):
        t = src_k[pl.ds(g * 16, 16)]
        ix = src_i[pl.ds(g * 16, 16)]
        m = (g * 16 + lane) < limit
        if check_sent:
          m = m & (t != SENT)
        d = jnp.clip(
            lax.shift_right_logical(t, shift) & (NBKT - 1), 0, NBKT - 1)
        occ, lastm = plsc.scan_count(d, m)
        cur = plsc.load_gather(run, [d], mask=m)
        pos = jnp.clip(cur + occ - 1, 0, KCAP - 1)
        plsc.store_scatter(dst_k, [pos], t, mask=m)
        plsc.store_scatter(dst_i, [pos], ix, mask=m)
        plsc.addupdate_scatter(run, [d], occ, mask=lastm & m)
        return 0

      lax.fori_loop(0, ng, dgrp, 0)

    ngr = (n_real + 15) // 16
    radix_pass(0, ka, ia, kb, ib, n_eff, ngrp, True)
    radix_pass(10, kb, ib, ka, ia, n_real, ngr, False)

    @pl.when(c1 > 0)
    def _pass3():
      radix_pass(20, ka, ia, kb, ib, n_real, ngr, False)

    def publish(src):
      def tkgrp(g, _):
        posv = g * 16 + lane
        ix = src[pl.ds(g * 16, 16)]
        ck[pl.ds(g * 16, 16)] = jnp.where(posv < n_real, ix, posv - n_real)
        return 0

      lax.fori_loop(0, TOP_K // 16, tkgrp, 0)

    @pl.when(c1 == 0)
    def _pub2():
      publish(ia)

    @pl.when(c1 > 0)
    def _pub3():
      publish(ib)

    pltpu.sync_copy(ck, spm_topk.at[h])

  @pl.when(strip == 0)
  def _sorter():
    with jax.named_scope("ph2_sort"):
      _sorter_body()

  plsc.subcore_barrier()

  band_cp.wait()
  pltpu.sync_copy(spm_topk.at[h], tk)

  def scan_grp(g, wcnt):
    ix = tk[pl.ds(g * 16, 16)]
    row = lax.shift_right_logical(ix, 9)
    m = (lax.shift_right_logical(row, 6) == strip) & (wcnt < WCAP - 16)
    occ = plsc.cumsum(jnp.where(m, 1, 0))
    pos = wcnt + occ - 1
    plsc.store_scatter(wl_pos, [pos], g * 16 + lane, mask=m)
    plsc.store_scatter(wl_idx, [pos], ix, mask=m)
    return wcnt + plsc.all_reduce_population_count(m)

  with jax.named_scope("ph3_scan"):
    wcnt = jnp.max(
        lax.fori_loop(0, TOP_K // 16, scan_grp, jnp.zeros((16,), jnp.int32)))

  dump = h * RES_HALF + 4 * TOP_K

  def rid_init(j, _):
    rid[pl.ds(j * 16, 16)] = dump + ((j * 16 + lane) & 63)
    return 0

  lax.fori_loop(0, WCAP * 4 // 16, rid_init, 0)

  offs = [(dy, dx) for dy in range(KSZ) for dx in range(KSZ)]

  def refine_grp(g, _):
    i_loc = g * 16 + lane
    m = i_loc < wcnt
    pos = wl_pos[pl.ds(g * 16, 16)]
    ix = wl_idx[pl.ds(g * 16, 16)]
    row = lax.shift_right_logical(ix, 9)
    col = ix & (W - 1)

    mx = jnp.zeros((16,), jnp.float32)
    for t, (dy, dx) in enumerate(offs):
      rr = row + (dy - RADIUS)
      cc = col + (dx - RADIUS)
      inb = m & (rr >= 0) & (rr < H) & (cc >= 0) & (cc < W)
      gidx = jnp.clip((rr - r0s) * W + cc, 0, BAND_ELEMS - 1)
      v = plsc.load_gather(band, [gidx], mask=inb)
      v = jnp.where(inb, v, 0.0)
      patch[t] = v
      mx = jnp.maximum(mx, v)

    s0 = jnp.zeros((16,), jnp.float32)
    sx = jnp.zeros((16,), jnp.float32)
    sy = jnp.zeros((16,), jnp.float32)
    s2 = jnp.zeros((16,), jnp.float32)
    for t, (dy, dx) in enumerate(offs):
      dxf = float(dx - RADIUS)
      dyf = float(dy - RADIUS)
      e = jnp.exp((patch[t] - mx) * (1.0 / TEMP))
      s0 = s0 + e
      sx = sx + e * dxf
      sy = sy + e * dyf
      s2 = s2 + e * (dxf * dxf + dyf * dyf)

    rx = sx / s0
    ry = sy / s0
    disp = (s2 / s0 - rx * rx - ry * ry) * (1.0 / (RADIUS * RADIUS))
    colf = col.astype(jnp.float32)
    rowf = row.astype(jnp.float32)
    kx = (colf + rx) / (W - 1) * 2.0 - 1.0
    ky = (rowf + ry) / (H - 1) * 2.0 - 1.0

    x = (kx + 1.0) * 0.5 * (W - 1)
    y = (ky + 1.0) * 0.5 * (H - 1)
    xt = x.astype(jnp.int32).astype(jnp.float32)
    x0f = jnp.where(x < xt, xt - 1.0, xt)
    yt = y.astype(jnp.int32).astype(jnp.float32)
    y0f = jnp.where(y < yt, yt - 1.0, yt)
    x1f = x0f + 1.0
    y1f = y0f + 1.0
    x0i = jnp.clip(x0f.astype(jnp.int32), 0, W - 1)
    x1i = jnp.clip(x1f.astype(jnp.int32), 0, W - 1)
    y0i = jnp.clip(y0f.astype(jnp.int32), 0, H - 1)
    y1i = jnp.clip(y1f.astype(jnp.int32), 0, H - 1)

    def samp(yi, xi):
      gi = jnp.clip((yi - r0s) * W + xi, 0, BAND_ELEMS - 1)
      v = plsc.load_gather(band, [gi], mask=m)
      return jnp.where(m, v, 0.0)

    wa = (x1f - x) * (y1f - y)
    wb = (x1f - x) * (y - y0f)
    wc = (x - x0f) * (y1f - y)
    wd = (x - x0f) * (y - y0f)
    ks = (wa * samp(y0i, x0i) + wb * samp(y1i, x0i)
          + wc * samp(y0i, x1i) + wd * samp(y1i, x1i))

    i4 = i_loc * 4
    p4 = h * RES_HALF + pos * 4
    for q, val in enumerate((kx, ky, ks, disp)):
      f = i4 + q
      plsc.store_scatter(st, [f], val, mask=m)
      plsc.store_scatter(rid, [f], p4 + q, mask=m)
    return 0

  with jax.named_scope("ph3_refine"):
    lax.fori_loop(0, (wcnt + 15) // 16, refine_grp, 0)

  with jax.named_scope("ph3_scatter"):
    for k in range(WCAP * 4 // 512):
      @pl.when(wcnt * 4 > k * 512)
      def _scat(k=k):
        def cp(j, _):
          rchunk[pl.ds(j * 16, 16)] = rid[pl.ds(k * 512 + j * 16, 16)]
          return 0

        lax.fori_loop(0, 32, cp, 0)
        pltpu.sync_copy(st.at[pl.ds(k * 512, 512)], spm_res.at[rchunk],
                        add=True)

  plsc.subcore_barrier()

  @pl.when(strip == 0)
  def _writeback():
    pltpu.sync_copy(spm_res.at[pl.ds(h * RES_HALF, 4 * TOP_K)], out_hbm.at[img])


@functools.partial(
    pl.kernel,
    out_type=jax.ShapeDtypeStruct((B, 4 * TOP_K), jnp.float32),
    mesh=plsc.VectorSubcoreMesh(core_axis_name="c", subcore_axis_name="s"),
    compiler_params=pltpu.CompilerParams(needs_layout_passes=False),
    scratch_types=[
        pltpu.VMEM((16384,), jnp.int32),
        pltpu.VMEM((CAP_STRIP,), jnp.int32),
        pltpu.VMEM((CAP_STRIP,), jnp.int32),
        pltpu.VMEM((NBKT,), jnp.int32),
        pltpu.VMEM((NBKT,), jnp.int32),
        pltpu.VMEM((KCAP,), jnp.int32),
        pltpu.VMEM((KCAP,), jnp.int32),
        pltpu.VMEM((KCAP,), jnp.int32),
        pltpu.VMEM((KCAP,), jnp.int32),
        pltpu.VMEM((16,), jnp.int32),
        pltpu.VMEM((BAND_ELEMS,), jnp.float32),
        pltpu.VMEM((TOP_K,), jnp.int32),
        pltpu.VMEM((WCAP,), jnp.int32),
        pltpu.VMEM((WCAP,), jnp.int32),
        pltpu.VMEM((25, 16), jnp.float32),
        pltpu.VMEM((WCAP * 4,), jnp.float32),
        pltpu.VMEM((WCAP * 4,), jnp.int32),
        pltpu.VMEM((512,), jnp.int32),
        pltpu.VMEM_SHARED((2, 8, KEEP_CAP), jnp.int32),
        pltpu.VMEM_SHARED((2, 8, KEEP_CAP), jnp.int32),
        pltpu.VMEM_SHARED((2, 128), jnp.int32),
        pltpu.VMEM_SHARED((2, 8 * NBKT), jnp.int32),
        pltpu.VMEM_SHARED((2, TOP_K), jnp.int32),
        pltpu.VMEM_SHARED((2 * RES_HALF,), jnp.float32),
        pltpu.SemaphoreType.DMA,
    ],
)
def _sc_call(nms_hbm, img_hbm, out_hbm, *scratch):
  _sc_body(nms_hbm, img_hbm, out_hbm, *scratch)


@jax.jit
def kernel(scores_map):
  s3 = scores_map.reshape(B, H, W)
  nms = _nms_call(s3)
  out = _sc_call(nms.reshape(B * HW), s3.reshape(B * HW))
  o = out.reshape(B, TOP_K, 4)
  return o[:, :, 0:2], o[:, :, 2], o[:, :, 3]


# speedup vs baseline: 1.4911x; 1.1211x over previous
import functools

import jax
import jax.numpy as jnp
from jax import lax
from jax.experimental import pallas as pl
from jax.experimental.pallas import tpu as pltpu
from jax.experimental.pallas import tpu_sc as plsc

B, H, W = 4, 512, 512
RADIUS = 2
TOP_K = 4096
TEMP = 0.1
KSZ = 2 * RADIUS + 1

HW = H * W
CAP_STRIP = 4096
KEEP_CAP = 1024
KCAP = 8192
WCAP = 1536
BAND_ROWS = 70
BAND_ELEMS = BAND_ROWS * W
NBKT = 1024
RES_HALF = 4 * TOP_K + 128
RES_SLICE = 2 * RES_HALF // 16
FLIP = 0x3F800000
SENT = 0x3FFFFFFF


def _maxpool5(x):
  neg_r = jnp.full((RADIUS, W), -1.0, jnp.float32)
  xp = jnp.concatenate([neg_r, x, neg_r], axis=0)
  r = xp[0:H]
  for i in range(1, KSZ):
    r = jnp.maximum(r, xp[i:i + H])
  neg_c = jnp.full((H, RADIUS), -1.0, jnp.float32)
  rp = jnp.concatenate([neg_c, r, neg_c], axis=1)
  c = rp[:, 0:W]
  for i in range(1, KSZ):
    c = jnp.maximum(c, rp[:, i:i + W])
  return c


def _nms_body(s_ref, o_ref):
  s = s_ref[0]
  max_mask = s == _maxpool5(s)
  for _ in range(2):
    supp = _maxpool5(max_mask.astype(jnp.float32)) > 0
    ss = jnp.where(supp, 0.0, s)
    new_max = ss == _maxpool5(ss)
    max_mask = max_mask | (new_max & (~supp))
  ri = lax.broadcasted_iota(jnp.int32, (H, W), 0)
  ci = lax.broadcasted_iota(jnp.int32, (H, W), 1)
  border = (ri >= RADIUS) & (ri < H - RADIUS) & (ci >= RADIUS) & (ci < W - RADIUS)
  bits = lax.bitcast_convert_type(s, jnp.int32)
  o_ref[0] = jnp.where(max_mask & border & (s > 0.0), FLIP - bits, FLIP)


def _nms_call(s3):
  return pl.pallas_call(
      _nms_body,
      grid=(B,),
      in_specs=[pl.BlockSpec((1, H, W), lambda b: (b, 0, 0))],
      out_specs=pl.BlockSpec((1, H, W), lambda b: (b, 0, 0)),
      out_shape=jax.ShapeDtypeStruct((B, H, W), jnp.int32),
  )(s3)


def _sc_body(nms_hbm, img_hbm, out_hbm,
             sbuf, ck, ci, hist, run, ka, ia, kb, ib, cntbuf,
             band, tk, wl_pos, wl_idx, patch, st, rid, rchunk,
             spm_kt, spm_ki, spm_cnt, spm_h1, spm_topk, spm_res, sem):
  c = lax.axis_index("c")
  s = lax.axis_index("s")
  h = s // 8
  strip = s % 8
  img = 2 * c + h
  lane = lax.iota(jnp.int32, 16)

  def zgrp(j, _):
    st[pl.ds(j * 16, 16)] = jnp.zeros((16,), jnp.float32)
    return 0

  lax.fori_loop(0, (RES_SLICE + 15) // 16, zgrp, 0)
  pltpu.sync_copy(st.at[pl.ds(0, RES_SLICE)],
                  spm_res.at[pl.ds(s * RES_SLICE, RES_SLICE)])

  r0s = jnp.minimum(jnp.maximum(64 * strip - 2, 0), H - BAND_ROWS)
  band_cp = pltpu.async_copy(
      img_hbm.at[pl.ds(img * HW + r0s * W, BAND_ELEMS)], band, sem)

  def zero_hist(ref):
    def z(j, _):
      ref[pl.ds(j * 16, 16)] = jnp.zeros((16,), jnp.int32)
      return 0
    lax.fori_loop(0, NBKT // 16, z, 0)

  def prefix_into_run(j, tot):
    hv = hist[pl.ds(j * 16, 16)]
    incl = plsc.cumsum(hv)
    run[pl.ds(j * 16, 16)] = tot + incl - hv
    return tot + jnp.max(incl)

  def merge_h1():
    def mj(j, _):
      acc = sbuf[pl.ds(2048 + j * 16, 16)]
      for stp in range(1, 8):
        acc = acc + sbuf[pl.ds(2048 + stp * NBKT + j * 16, 16)]
      hist[pl.ds(j * 16, 16)] = acc
      return 0
    lax.fori_loop(0, NBKT // 16, mj, 0)

  def compact_half(c2, cnt):
    base_off = img * HW + strip * 32768 + c2 * 16384
    pltpu.sync_copy(nms_hbm.at[pl.ds(base_off, 16384)], sbuf)

    def grp(g, cnt):
      t = sbuf[pl.ds(g * 16, 16)]
      m = (t < FLIP) & (cnt < CAP_STRIP - 16)
      pix = strip * 32768 + c2 * 16384 + g * 16 + lane
      occ = plsc.cumsum(jnp.where(m, 1, 0))
      pos = cnt + occ - 1
      plsc.store_scatter(ck, [pos], t, mask=m)
      plsc.store_scatter(ci, [pos], pix, mask=m)
      return cnt + plsc.all_reduce_population_count(m)

    return plsc.parallel_loop(0, 1024, unroll=4, carry=cnt)(grp)

  with jax.named_scope("ph1_compact"):
    cnt = lax.fori_loop(0, 2, compact_half, jnp.zeros((16,), jnp.int32))
  cs = jnp.max(cnt)
  csg = (cs + 15) // 16

  with jax.named_scope("ph1_hist"):
    zero_hist(hist)

    def h1grp(g, _):
      t = ck[pl.ds(g * 16, 16)]
      m = (g * 16 + lane) < cs
      bkt = jnp.clip(lax.shift_right_logical(t, 20), 0, NBKT - 1)
      occ, lastm = plsc.scan_count(bkt, m)
      plsc.addupdate_scatter(hist, [bkt], occ, mask=lastm & m)
      return 0

    lax.fori_loop(0, csg, h1grp, 0)
  pltpu.sync_copy(hist, spm_h1.at[h, pl.ds(strip * NBKT, NBKT)])

  plsc.subcore_barrier()

  with jax.named_scope("ph15_merge"):
    pltpu.sync_copy(spm_h1.at[h], sbuf.at[pl.ds(2048, 8 * NBKT)])
    merge_h1()
    lax.fori_loop(0, NBKT // 16, prefix_into_run, jnp.int32(0))

    def cntc(j, acc):
      exv = run[pl.ds(j * 16, 16)]
      return acc + plsc.all_reduce_population_count(exv < TOP_K)

    c1 = jnp.max(
        lax.fori_loop(0, NBKT // 16, cntc, jnp.zeros((16,), jnp.int32))) - 1

    def bsum(j, acc):
      hv = hist[pl.ds(j * 16, 16)]
      mv = (j * 16 + lane) < c1
      return acc + jnp.sum(jnp.where(mv, hv, 0))

    base1 = lax.fori_loop(0, NBKT // 16, bsum, jnp.int32(0))

  plsc.subcore_barrier()

  with jax.named_scope("ph15_hist2"):
    zero_hist(hist)

    def h2grp(g, _):
      t = ck[pl.ds(g * 16, 16)]
      m = (g * 16 + lane) < cs
      bkt = jnp.clip(lax.shift_right_logical(t, 20), 0, NBKT - 1)
      b2 = lax.shift_right_logical(t, 10) & (NBKT - 1)
      m2 = m & (bkt == c1)
      occ, lastm = plsc.scan_count(b2, m2)
      plsc.addupdate_scatter(hist, [b2], occ, mask=lastm & m2)
      return 0

    lax.fori_loop(0, csg, h2grp, 0)
  pltpu.sync_copy(hist, spm_h1.at[h, pl.ds(strip * NBKT, NBKT)])

  plsc.subcore_barrier()

  with jax.named_scope("ph16_keep"):
    pltpu.sync_copy(spm_h1.at[h], sbuf.at[pl.ds(2048, 8 * NBKT)])
    merge_h1()
    lax.fori_loop(0, NBKT // 16, prefix_into_run, base1)

    def sfill(j, _):
      sbuf[pl.ds(j * 16, 16)] = jnp.zeros((16,), jnp.int32) + SENT
      sbuf[pl.ds(KEEP_CAP + j * 16, 16)] = jnp.zeros((16,), jnp.int32)
      return 0

    lax.fori_loop(0, KEEP_CAP // 16, sfill, 0)

    def kgrp(g, kcnt):
      t = ck[pl.ds(g * 16, 16)]
      ix = ci[pl.ds(g * 16, 16)]
      m = (g * 16 + lane) < cs
      bkt = jnp.clip(lax.shift_right_logical(t, 20), 0, NBKT - 1)
      b2 = lax.shift_right_logical(t, 10) & (NBKT - 1)
      ex2 = plsc.load_gather(run, [b2], mask=m)
      keep = m & ((bkt < c1) | ((bkt == c1) & (ex2 < TOP_K))) & (
          kcnt < KEEP_CAP - 16)
      occ = plsc.cumsum(jnp.where(keep, 1, 0))
      pos = kcnt + occ - 1
      plsc.store_scatter(sbuf, [pos], t, mask=keep)
      plsc.store_scatter(sbuf, [KEEP_CAP + pos], ix, mask=keep)
      return kcnt + plsc.all_reduce_population_count(keep)

    kcnt = lax.fori_loop(0, csg, kgrp, jnp.zeros((16,), jnp.int32))
    cntbuf[...] = kcnt
  pltpu.sync_copy(sbuf.at[pl.ds(0, KEEP_CAP)], spm_kt.at[h, strip])
  pltpu.sync_copy(sbuf.at[pl.ds(KEEP_CAP, KEEP_CAP)], spm_ki.at[h, strip])
  pltpu.sync_copy(cntbuf, spm_cnt.at[h, pl.ds(strip * 16, 16)])

  plsc.subcore_barrier()

  def _sorter_body():
    pltpu.sync_copy(spm_cnt.at[h], sbuf.at[pl.ds(10240, 128)])
    off = jnp.int32(0)
    n_real = jnp.int32(0)
    n_eff = jnp.int32(0)
    for stp in range(8):
      kc = jnp.minimum(jnp.max(sbuf[pl.ds(10240 + stp * 16, 16)]), KEEP_CAP)
      pltpu.sync_copy(spm_kt.at[h, stp], ka.at[pl.ds(off, KEEP_CAP)])
      pltpu.sync_copy(spm_ki.at[h, stp], ia.at[pl.ds(off, KEEP_CAP)])
      if stp == 7:
        n_eff = off + KEEP_CAP
      off = off + ((kc + 127) // 128) * 128
      n_real = n_real + kc
    ngrp = (n_eff + 15) // 16

    def radix_pass(shift, src_k, src_i, dst_k, dst_i, limit, ng, check_sent):
      zero_hist(hist)

      def hgrp(g, _):
        t = src_k[pl.ds(g * 16, 16)]
        m = (g * 16 + lane) < limit
        if check_sent:
          m = m & (t != SENT)
        d = jnp.clip(
            lax.shift_right_logical(t, shift) & (NBKT - 1), 0, NBKT - 1)
        occ, lastm = plsc.scan_count(d, m)
        plsc.addupdate_scatter(hist, [d], occ, mask=lastm & m)
        return 0

      lax.fori_loop(0, ng, hgrp, 0)
      lax.fori_loop(0, NBKT // 16, prefix_into_run, jnp.int32(0))

      def dgrp(g, _):
        t = src_k[pl.ds(g * 16, 16)]
        ix = src_i[pl.ds(g * 16, 16)]
        m = (g * 16 + lane) < limit
        if check_sent:
          m = m & (t != SENT)
        d = jnp.clip(
            lax.shift_right_logical(t, shift) & (NBKT - 1), 0, NBKT - 1)
        occ, lastm = plsc.scan_count(d, m)
        cur = plsc.load_gather(run, [d], mask=m)
        pos = jnp.clip(cur + occ - 1, 0, KCAP - 1)
        plsc.store_scatter(dst_k, [pos], t, mask=m)
        plsc.store_scatter(dst_i, [pos], ix, mask=m)
        plsc.addupdate_scatter(run, [d], occ, mask=lastm & m)
        return 0

      lax.fori_loop(0, ng, dgrp, 0)

    ngr = (n_real + 15) // 16
    radix_pass(0, ka, ia, kb, ib, n_eff, ngrp, True)
    radix_pass(10, kb, ib, ka, ia, n_real, ngr, False)

    @pl.when(c1 > 0)
    def _pass3():
      radix_pass(20, ka, ia, kb, ib, n_real, ngr, False)

    def publish(src):
      def tkgrp(g, _):
        posv = g * 16 + lane
        ix = src[pl.ds(g * 16, 16)]
        ck[pl.ds(g * 16, 16)] = jnp.where(posv < n_real, ix, posv - n_real)
        return 0

      lax.fori_loop(0, TOP_K // 16, tkgrp, 0)

    @pl.when(c1 == 0)
    def _pub2():
      publish(ia)

    @pl.when(c1 > 0)
    def _pub3():
      publish(ib)

    pltpu.sync_copy(ck, spm_topk.at[h])

  @pl.when(strip == 0)
  def _sorter():
    with jax.named_scope("ph2_sort"):
      _sorter_body()

  plsc.subcore_barrier()

  band_cp.wait()
  pltpu.sync_copy(spm_topk.at[h], tk)

  def scan_grp(g, wcnt):
    ix = tk[pl.ds(g * 16, 16)]
    row = lax.shift_right_logical(ix, 9)
    m = (lax.shift_right_logical(row, 6) == strip) & (wcnt < WCAP - 16)
    occ = plsc.cumsum(jnp.where(m, 1, 0))
    pos = wcnt + occ - 1
    plsc.store_scatter(wl_pos, [pos], g * 16 + lane, mask=m)
    plsc.store_scatter(wl_idx, [pos], ix, mask=m)
    return wcnt + plsc.all_reduce_population_count(m)

  with jax.named_scope("ph3_scan"):
    wcnt = jnp.max(
        lax.fori_loop(0, TOP_K // 16, scan_grp, jnp.zeros((16,), jnp.int32)))

  dump = h * RES_HALF + 4 * TOP_K

  def rid_init(j, _):
    rid[pl.ds(j * 16, 16)] = dump + ((j * 16 + lane) & 63)
    return 0

  lax.fori_loop(0, WCAP * 4 // 16, rid_init, 0)

  offs = [(dy, dx) for dy in range(KSZ) for dx in range(KSZ)]

  def refine_grp(g, _):
    i_loc = g * 16 + lane
    m = i_loc < wcnt
    pos = wl_pos[pl.ds(g * 16, 16)]
    ix = wl_idx[pl.ds(g * 16, 16)]
    row = lax.shift_right_logical(ix, 9)
    col = ix & (W - 1)

    mx = jnp.zeros((16,), jnp.float32)
    for t, (dy, dx) in enumerate(offs):
      rr = row + (dy - RADIUS)
      cc = col + (dx - RADIUS)
      inb = m & (rr >= 0) & (rr < H) & (cc >= 0) & (cc < W)
      gidx = jnp.clip((rr - r0s) * W + cc, 0, BAND_ELEMS - 1)
      v = plsc.load_gather(band, [gidx], mask=inb)
      v = jnp.where(inb, v, 0.0)
      patch[t] = v
      mx = jnp.maximum(mx, v)

    s0 = jnp.zeros((16,), jnp.float32)
    sx = jnp.zeros((16,), jnp.float32)
    sy = jnp.zeros((16,), jnp.float32)
    s2 = jnp.zeros((16,), jnp.float32)
    for t, (dy, dx) in enumerate(offs):
      dxf = float(dx - RADIUS)
      dyf = float(dy - RADIUS)
      e = jnp.exp((patch[t] - mx) * (1.0 / TEMP))
      s0 = s0 + e
      sx = sx + e * dxf
      sy = sy + e * dyf
      s2 = s2 + e * (dxf * dxf + dyf * dyf)

    rx = sx / s0
    ry = sy / s0
    disp = (s2 / s0 - rx * rx - ry * ry) * (1.0 / (RADIUS * RADIUS))
    colf = col.astype(jnp.float32)
    rowf = row.astype(jnp.float32)
    kx = (colf + rx) / (W - 1) * 2.0 - 1.0
    ky = (rowf + ry) / (H - 1) * 2.0 - 1.0

    x = (kx + 1.0) * 0.5 * (W - 1)
    y = (ky + 1.0) * 0.5 * (H - 1)
    xt = x.astype(jnp.int32).astype(jnp.float32)
    x0f = jnp.where(x < xt, xt - 1.0, xt)
    yt = y.astype(jnp.int32).astype(jnp.float32)
    y0f = jnp.where(y < yt, yt - 1.0, yt)
    x1f = x0f + 1.0
    y1f = y0f + 1.0
    x0i = jnp.clip(x0f.astype(jnp.int32), 0, W - 1)
    x1i = jnp.clip(x1f.astype(jnp.int32), 0, W - 1)
    y0i = jnp.clip(y0f.astype(jnp.int32), 0, H - 1)
    y1i = jnp.clip(y1f.astype(jnp.int32), 0, H - 1)

    def samp(yi, xi):
      gi = jnp.clip((yi - r0s) * W + xi, 0, BAND_ELEMS - 1)
      v = plsc.load_gather(band, [gi], mask=m)
      return jnp.where(m, v, 0.0)

    wa = (x1f - x) * (y1f - y)
    wb = (x1f - x) * (y - y0f)
    wc = (x - x0f) * (y1f - y)
    wd = (x - x0f) * (y - y0f)
    ks = (wa * samp(y0i, x0i) + wb * samp(y1i, x0i)
          + wc * samp(y0i, x1i) + wd * samp(y1i, x1i))

    i4 = i_loc * 4
    p4 = h * RES_HALF + pos * 4
    for q, val in enumerate((kx, ky, ks, disp)):
      f = i4 + q
      plsc.store_scatter(st, [f], val, mask=m)
      plsc.store_scatter(rid, [f], p4 + q, mask=m)
    return 0

  with jax.named_scope("ph3_refine"):
    lax.fori_loop(0, (wcnt + 15) // 16, refine_grp, 0)

  with jax.named_scope("ph3_scatter"):
    for k in range(WCAP * 4 // 512):
      @pl.when(wcnt * 4 > k * 512)
      def _scat(k=k):
        def cp(j, _):
          rchunk[pl.ds(j * 16, 16)] = rid[pl.ds(k * 512 + j * 16, 16)]
          return 0

        lax.fori_loop(0, 32, cp, 0)
        pltpu.sync_copy(st.at[pl.ds(k * 512, 512)], spm_res.at[rchunk],
                        add=True)

  plsc.subcore_barrier()

  @pl.when(strip == 0)
  def _writeback():
    pltpu.sync_copy(spm_res.at[pl.ds(h * RES_HALF, 4 * TOP_K)], out_hbm.at[img])


@functools.partial(
    pl.kernel,
    out_type=jax.ShapeDtypeStruct((B, 4 * TOP_K), jnp.float32),
    mesh=plsc.VectorSubcoreMesh(core_axis_name="c", subcore_axis_name="s"),
    compiler_params=pltpu.CompilerParams(needs_layout_passes=False),
    scratch_types=[
        pltpu.VMEM((16384,), jnp.int32),
        pltpu.VMEM((CAP_STRIP,), jnp.int32),
        pltpu.VMEM((CAP_STRIP,), jnp.int32),
        pltpu.VMEM((NBKT,), jnp.int32),
        pltpu.VMEM((NBKT,), jnp.int32),
        pltpu.VMEM((KCAP,), jnp.int32),
        pltpu.VMEM((KCAP,), jnp.int32),
        pltpu.VMEM((KCAP,), jnp.int32),
        pltpu.VMEM((KCAP,), jnp.int32),
        pltpu.VMEM((16,), jnp.int32),
        pltpu.VMEM((BAND_ELEMS,), jnp.float32),
        pltpu.VMEM((TOP_K,), jnp.int32),
        pltpu.VMEM((WCAP,), jnp.int32),
        pltpu.VMEM((WCAP,), jnp.int32),
        pltpu.VMEM((25, 16), jnp.float32),
        pltpu.VMEM((WCAP * 4,), jnp.float32),
        pltpu.VMEM((WCAP * 4,), jnp.int32),
        pltpu.VMEM((512,), jnp.int32),
        pltpu.VMEM_SHARED((2, 8, KEEP_CAP), jnp.int32),
        pltpu.VMEM_SHARED((2, 8, KEEP_CAP), jnp.int32),
        pltpu.VMEM_SHARED((2, 128), jnp.int32),
        pltpu.VMEM_SHARED((2, 8 * NBKT), jnp.int32),
        pltpu.VMEM_SHARED((2, TOP_K), jnp.int32),
        pltpu.VMEM_SHARED((2 * RES_HALF,), jnp.float32),
        pltpu.SemaphoreType.DMA,
    ],
)
def _sc_call(nms_hbm, img_hbm, out_hbm, *scratch):
  _sc_body(nms_hbm, img_hbm, out_hbm, *scratch)


@jax.jit
def kernel(scores_map):
  s3 = scores_map.reshape(B, H, W)
  nms = _nms_call(s3)
  out = _sc_call(nms.reshape(B * HW), s3.reshape(B * HW))
  o = out.reshape(B, TOP_K, 4)
  return o[:, :, 0:2], o[:, :, 2], o[:, :, 3]


# speedup vs baseline: 1.5397x; 1.0326x over previous
import functools

import jax
import jax.numpy as jnp
from jax import lax
from jax.experimental import pallas as pl
from jax.experimental.pallas import tpu as pltpu
from jax.experimental.pallas import tpu_sc as plsc

B, H, W = 4, 512, 512
RADIUS = 2
TOP_K = 4096
TEMP = 0.1
KSZ = 2 * RADIUS + 1

HW = H * W
CAP_STRIP = 4096
KEEP_CAP = 1024
KCAP = 8192
WCAP = 1536
BAND_ROWS = 70
BAND_ELEMS = BAND_ROWS * W
NBKT = 1024
RES_HALF = 4 * TOP_K + 128
RES_SLICE = 2 * RES_HALF // 16
FLIP = 0x3F800000
SENT = 0x3FFFFFFF


def _maxpool5(x):
  neg_r = jnp.full((RADIUS, W), -1.0, jnp.float32)
  xp = jnp.concatenate([neg_r, x, neg_r], axis=0)
  r = xp[0:H]
  for i in range(1, KSZ):
    r = jnp.maximum(r, xp[i:i + H])
  neg_c = jnp.full((H, RADIUS), -1.0, jnp.float32)
  rp = jnp.concatenate([neg_c, r, neg_c], axis=1)
  c = rp[:, 0:W]
  for i in range(1, KSZ):
    c = jnp.maximum(c, rp[:, i:i + W])
  return c


def _nms_body(s_ref, o_ref):
  s = s_ref[0]
  max_mask = s == _maxpool5(s)
  for _ in range(2):
    supp = _maxpool5(max_mask.astype(jnp.float32)) > 0
    ss = jnp.where(supp, 0.0, s)
    new_max = ss == _maxpool5(ss)
    max_mask = max_mask | (new_max & (~supp))
  ri = lax.broadcasted_iota(jnp.int32, (H, W), 0)
  ci = lax.broadcasted_iota(jnp.int32, (H, W), 1)
  border = (ri >= RADIUS) & (ri < H - RADIUS) & (ci >= RADIUS) & (ci < W - RADIUS)
  bits = lax.bitcast_convert_type(s, jnp.int32)
  o_ref[0] = jnp.where(max_mask & border & (s > 0.0), FLIP - bits, FLIP)


def _nms_call(s3):
  return pl.pallas_call(
      _nms_body,
      grid=(B,),
      in_specs=[pl.BlockSpec((1, H, W), lambda b: (b, 0, 0))],
      out_specs=pl.BlockSpec((1, H, W), lambda b: (b, 0, 0)),
      out_shape=jax.ShapeDtypeStruct((B, H, W), jnp.int32),
  )(s3)


def _sc_body(nms_hbm, img_hbm, out_hbm,
             sbuf, ck, ci, hist, run, ka, ia, kb, ib, cntbuf,
             band, tk, wl_pos, wl_idx, patch, st, rid, rchunk,
             spm_kt, spm_ki, spm_cnt, spm_h1, spm_topk, spm_res, sem):
  c = lax.axis_index("c")
  s = lax.axis_index("s")
  h = s // 8
  strip = s % 8
  img = 2 * c + h
  lane = lax.iota(jnp.int32, 16)

  def zgrp(j, _):
    st[pl.ds(j * 16, 16)] = jnp.zeros((16,), jnp.float32)
    return 0

  lax.fori_loop(0, (RES_SLICE + 15) // 16, zgrp, 0)
  pltpu.sync_copy(st.at[pl.ds(0, RES_SLICE)],
                  spm_res.at[pl.ds(s * RES_SLICE, RES_SLICE)])

  r0s = jnp.minimum(jnp.maximum(64 * strip - 2, 0), H - BAND_ROWS)
  band_cp = pltpu.async_copy(
      img_hbm.at[pl.ds(img * HW + r0s * W, BAND_ELEMS)], band, sem)

  def zero_hist(ref):
    def z(j, _):
      ref[pl.ds(j * 16, 16)] = jnp.zeros((16,), jnp.int32)
      return 0
    lax.fori_loop(0, NBKT // 16, z, 0)

  def prefix_into_run(j, tot):
    hv = hist[pl.ds(j * 16, 16)]
    incl = plsc.cumsum(hv)
    run[pl.ds(j * 16, 16)] = tot + incl - hv
    return tot + jnp.max(incl)

  def merge_h1():
    def mj(j, _):
      acc = sbuf[pl.ds(2048 + j * 16, 16)]
      for stp in range(1, 8):
        acc = acc + sbuf[pl.ds(2048 + stp * NBKT + j * 16, 16)]
      hist[pl.ds(j * 16, 16)] = acc
      return 0
    lax.fori_loop(0, NBKT // 16, mj, 0)

  def compact_half(c2, cnt):
    base_off = img * HW + strip * 32768 + c2 * 16384
    pltpu.sync_copy(nms_hbm.at[pl.ds(base_off, 16384)], sbuf)

    def grp(g, cnt):
      t = sbuf[pl.ds(g * 16, 16)]
      m = (t < FLIP) & (cnt < CAP_STRIP - 16)
      pix = strip * 32768 + c2 * 16384 + g * 16 + lane
      occ = plsc.cumsum(jnp.where(m, 1, 0))
      pos = cnt + occ - 1
      plsc.store_scatter(ck, [pos], t, mask=m)
      plsc.store_scatter(ci, [pos], pix, mask=m)
      return cnt + plsc.all_reduce_population_count(m)

    return plsc.parallel_loop(0, 1024, unroll=4, carry=cnt)(grp)

  with jax.named_scope("ph1_compact"):
    cnt = lax.fori_loop(0, 2, compact_half, jnp.zeros((16,), jnp.int32))
  cs = jnp.max(cnt)
  csg = (cs + 15) // 16

  with jax.named_scope("ph1_hist"):
    zero_hist(hist)

    def h1grp(g, _):
      t = ck[pl.ds(g * 16, 16)]
      m = (g * 16 + lane) < cs
      bkt = jnp.clip(lax.shift_right_logical(t, 20), 0, NBKT - 1)
      occ, lastm = plsc.scan_count(bkt, m)
      plsc.addupdate_scatter(hist, [bkt], occ, mask=lastm & m)
      return 0

    lax.fori_loop(0, csg, h1grp, 0)
  pltpu.sync_copy(hist, spm_h1.at[h, pl.ds(strip * NBKT, NBKT)])

  plsc.subcore_barrier()

  with jax.named_scope("ph15_merge"):
    pltpu.sync_copy(spm_h1.at[h], sbuf.at[pl.ds(2048, 8 * NBKT)])
    merge_h1()
    lax.fori_loop(0, NBKT // 16, prefix_into_run, jnp.int32(0))

    def cntc(j, acc):
      exv = run[pl.ds(j * 16, 16)]
      return acc + plsc.all_reduce_population_count(exv < TOP_K)

    c1 = jnp.max(
        lax.fori_loop(0, NBKT // 16, cntc, jnp.zeros((16,), jnp.int32))) - 1

    def bsum(j, acc):
      hv = hist[pl.ds(j * 16, 16)]
      mv = (j * 16 + lane) < c1
      return acc + jnp.sum(jnp.where(mv, hv, 0))

    base1 = lax.fori_loop(0, NBKT // 16, bsum, jnp.int32(0))

  plsc.subcore_barrier()

  with jax.named_scope("ph15_hist2"):
    zero_hist(hist)

    def h2grp(g, _):
      t = ck[pl.ds(g * 16, 16)]
      m = (g * 16 + lane) < cs
      bkt = jnp.clip(lax.shift_right_logical(t, 20), 0, NBKT - 1)
      b2 = lax.shift_right_logical(t, 10) & (NBKT - 1)
      m2 = m & (bkt == c1)
      occ, lastm = plsc.scan_count(b2, m2)
      plsc.addupdate_scatter(hist, [b2], occ, mask=lastm & m2)
      return 0

    lax.fori_loop(0, csg, h2grp, 0)
  pltpu.sync_copy(hist, spm_h1.at[h, pl.ds(strip * NBKT, NBKT)])

  plsc.subcore_barrier()

  with jax.named_scope("ph16_keep"):
    pltpu.sync_copy(spm_h1.at[h], sbuf.at[pl.ds(2048, 8 * NBKT)])
    merge_h1()
    lax.fori_loop(0, NBKT // 16, prefix_into_run, base1)

    def sfill(j, _):
      sbuf[pl.ds(j * 16, 16)] = jnp.zeros((16,), jnp.int32) + SENT
      sbuf[pl.ds(KEEP_CAP + j * 16, 16)] = jnp.zeros((16,), jnp.int32)
      return 0

    lax.fori_loop(0, KEEP_CAP // 16, sfill, 0)

    def kgrp(g, kcnt):
      t = ck[pl.ds(g * 16, 16)]
      ix = ci[pl.ds(g * 16, 16)]
      m = (g * 16 + lane) < cs
      bkt = jnp.clip(lax.shift_right_logical(t, 20), 0, NBKT - 1)
      b2 = lax.shift_right_logical(t, 10) & (NBKT - 1)
      ex2 = plsc.load_gather(run, [b2], mask=m)
      keep = m & ((bkt < c1) | ((bkt == c1) & (ex2 < TOP_K))) & (
          kcnt < KEEP_CAP - 16)
      occ = plsc.cumsum(jnp.where(keep, 1, 0))
      pos = kcnt + occ - 1
      plsc.store_scatter(sbuf, [pos], t, mask=keep)
      plsc.store_scatter(sbuf, [KEEP_CAP + pos], ix, mask=keep)
      return kcnt + plsc.all_reduce_population_count(keep)

    kcnt = plsc.parallel_loop(
        0, csg, unroll=4, carry=jnp.zeros((16,), jnp.int32))(kgrp)
    cntbuf[...] = kcnt
  pltpu.sync_copy(sbuf.at[pl.ds(0, KEEP_CAP)], spm_kt.at[h, strip])
  pltpu.sync_copy(sbuf.at[pl.ds(KEEP_CAP, KEEP_CAP)], spm_ki.at[h, strip])
  pltpu.sync_copy(cntbuf, spm_cnt.at[h, pl.ds(strip * 16, 16)])

  plsc.subcore_barrier()

  def _sorter_body():
    pltpu.sync_copy(spm_cnt.at[h], sbuf.at[pl.ds(10240, 128)])
    off = jnp.int32(0)
    n_real = jnp.int32(0)
    n_eff = jnp.int32(0)
    for stp in range(8):
      kc = jnp.minimum(jnp.max(sbuf[pl.ds(10240 + stp * 16, 16)]), KEEP_CAP)
      pltpu.sync_copy(spm_kt.at[h, stp], ka.at[pl.ds(off, KEEP_CAP)])
      pltpu.sync_copy(spm_ki.at[h, stp], ia.at[pl.ds(off, KEEP_CAP)])
      if stp == 7:
        n_eff = off + KEEP_CAP
      off = off + ((kc + 127) // 128) * 128
      n_real = n_real + kc
    ngrp = (n_eff + 15) // 16

    def radix_pass(shift, src_k, src_i, dst_k, dst_i, limit, ng, check_sent):
      zero_hist(hist)

      def hgrp(g, _):
        t = src_k[pl.ds(g * 16, 16)]
        m = (g * 16 + lane) < limit
        if check_sent:
          m = m & (t != SENT)
        d = jnp.clip(
            lax.shift_right_logical(t, shift) & (NBKT - 1), 0, NBKT - 1)
        occ, lastm = plsc.scan_count(d, m)
        plsc.addupdate_scatter(hist, [d], occ, mask=lastm & m)
        return 0

      lax.fori_loop(0, ng, hgrp, 0)
      lax.fori_loop(0, NBKT // 16, prefix_into_run, jnp.int32(0))

      def dgrp(g, _):
        t = src_k[pl.ds(g * 16, 16)]
        ix = src_i[pl.ds(g * 16, 16)]
        m = (g * 16 + lane) < limit
        if check_sent:
          m = m & (t != SENT)
        d = jnp.clip(
            lax.shift_right_logical(t, shift) & (NBKT - 1), 0, NBKT - 1)
        occ, lastm = plsc.scan_count(d, m)
        cur = plsc.load_gather(run, [d], mask=m)
        pos = jnp.clip(cur + occ - 1, 0, KCAP - 1)
        plsc.store_scatter(dst_k, [pos], t, mask=m)
        plsc.store_scatter(dst_i, [pos], ix, mask=m)
        plsc.addupdate_scatter(run, [d], occ, mask=lastm & m)
        return 0

      lax.fori_loop(0, ng, dgrp, 0)

    ngr = (n_real + 15) // 16
    radix_pass(0, ka, ia, kb, ib, n_eff, ngrp, True)
    radix_pass(10, kb, ib, ka, ia, n_real, ngr, False)

    @pl.when(c1 > 0)
    def _pass3():
      radix_pass(20, ka, ia, kb, ib, n_real, ngr, False)

    def publish(src):
      def tkgrp(g, _):
        posv = g * 16 + lane
        ix = src[pl.ds(g * 16, 16)]
        ck[pl.ds(g * 16, 16)] = jnp.where(posv < n_real, ix, posv - n_real)
        return 0

      lax.fori_loop(0, TOP_K // 16, tkgrp, 0)

    @pl.when(c1 == 0)
    def _pub2():
      publish(ia)

    @pl.when(c1 > 0)
    def _pub3():
      publish(ib)

    pltpu.sync_copy(ck, spm_topk.at[h])

  @pl.when(strip == 0)
  def _sorter():
    with jax.named_scope("ph2_sort"):
      _sorter_body()

  plsc.subcore_barrier()

  band_cp.wait()
  pltpu.sync_copy(spm_topk.at[h], tk)

  def scan_grp(g, wcnt):
    ix = tk[pl.ds(g * 16, 16)]
    row = lax.shift_right_logical(ix, 9)
    m = (lax.shift_right_logical(row, 6) == strip) & (wcnt < WCAP - 16)
    occ = plsc.cumsum(jnp.where(m, 1, 0))
    pos = wcnt + occ - 1
    plsc.store_scatter(wl_pos, [pos], g * 16 + lane, mask=m)
    plsc.store_scatter(wl_idx, [pos], ix, mask=m)
    return wcnt + plsc.all_reduce_population_count(m)

  with jax.named_scope("ph3_scan"):
    wcnt = jnp.max(
        plsc.parallel_loop(0, TOP_K // 16, unroll=4,
                           carry=jnp.zeros((16,), jnp.int32))(scan_grp))

  dump = h * RES_HALF + 4 * TOP_K

  def rid_init(j, _):
    rid[pl.ds(j * 16, 16)] = dump + ((j * 16 + lane) & 63)
    return 0

  lax.fori_loop(0, WCAP * 4 // 16, rid_init, 0)

  offs = [(dy, dx) for dy in range(KSZ) for dx in range(KSZ)]

  def refine_grp(g, _):
    i_loc = g * 16 + lane
    m = i_loc < wcnt
    pos = wl_pos[pl.ds(g * 16, 16)]
    ix = wl_idx[pl.ds(g * 16, 16)]
    row = lax.shift_right_logical(ix, 9)
    col = ix & (W - 1)

    mx = jnp.zeros((16,), jnp.float32)
    for t, (dy, dx) in enumerate(offs):
      rr = row + (dy - RADIUS)
      cc = col + (dx - RADIUS)
      inb = m & (rr >= 0) & (rr < H) & (cc >= 0) & (cc < W)
      gidx = jnp.clip((rr - r0s) * W + cc, 0, BAND_ELEMS - 1)
      v = plsc.load_gather(band, [gidx], mask=inb)
      v = jnp.where(inb, v, 0.0)
      patch[t] = v
      mx = jnp.maximum(mx, v)

    s0 = jnp.zeros((16,), jnp.float32)
    sx = jnp.zeros((16,), jnp.float32)
    sy = jnp.zeros((16,), jnp.float32)
    s2 = jnp.zeros((16,), jnp.float32)
    for t, (dy, dx) in enumerate(offs):
      dxf = float(dx - RADIUS)
      dyf = float(dy - RADIUS)
      e = jnp.exp((patch[t] - mx) * (1.0 / TEMP))
      s0 = s0 + e
      sx = sx + e * dxf
      sy = sy + e * dyf
      s2 = s2 + e * (dxf * dxf + dyf * dyf)

    rx = sx / s0
    ry = sy / s0
    disp = (s2 / s0 - rx * rx - ry * ry) * (1.0 / (RADIUS * RADIUS))
    colf = col.astype(jnp.float32)
    rowf = row.astype(jnp.float32)
    kx = (colf + rx) / (W - 1) * 2.0 - 1.0
    ky = (rowf + ry) / (H - 1) * 2.0 - 1.0

    x = (kx + 1.0) * 0.5 * (W - 1)
    y = (ky + 1.0) * 0.5 * (H - 1)
    xt = x.astype(jnp.int32).astype(jnp.float32)
    x0f = jnp.where(x < xt, xt - 1.0, xt)
    yt = y.astype(jnp.int32).astype(jnp.float32)
    y0f = jnp.where(y < yt, yt - 1.0, yt)
    x1f = x0f + 1.0
    y1f = y0f + 1.0
    x0i = jnp.clip(x0f.astype(jnp.int32), 0, W - 1)
    x1i = jnp.clip(x1f.astype(jnp.int32), 0, W - 1)
    y0i = jnp.clip(y0f.astype(jnp.int32), 0, H - 1)
    y1i = jnp.clip(y1f.astype(jnp.int32), 0, H - 1)

    def samp(yi, xi):
      gi = jnp.clip((yi - r0s) * W + xi, 0, BAND_ELEMS - 1)
      v = plsc.load_gather(band, [gi], mask=m)
      return jnp.where(m, v, 0.0)

    wa = (x1f - x) * (y1f - y)
    wb = (x1f - x) * (y - y0f)
    wc = (x - x0f) * (y1f - y)
    wd = (x - x0f) * (y - y0f)
    ks = (wa * samp(y0i, x0i) + wb * samp(y1i, x0i)
          + wc * samp(y0i, x1i) + wd * samp(y1i, x1i))

    i4 = i_loc * 4
    p4 = h * RES_HALF + pos * 4
    for q, val in enumerate((kx, ky, ks, disp)):
      f = i4 + q
      plsc.store_scatter(st, [f], val, mask=m)
      plsc.store_scatter(rid, [f], p4 + q, mask=m)
    return 0

  with jax.named_scope("ph3_refine"):
    lax.fori_loop(0, (wcnt + 15) // 16, refine_grp, 0)

  with jax.named_scope("ph3_scatter"):
    for k in range(WCAP * 4 // 512):
      @pl.when(wcnt * 4 > k * 512)
      def _scat(k=k):
        def cp(j, _):
          rchunk[pl.ds(j * 16, 16)] = rid[pl.ds(k * 512 + j * 16, 16)]
          return 0

        lax.fori_loop(0, 32, cp, 0)
        pltpu.sync_copy(st.at[pl.ds(k * 512, 512)], spm_res.at[rchunk],
                        add=True)

  plsc.subcore_barrier()

  @pl.when(strip == 0)
  def _writeback():
    pltpu.sync_copy(spm_res.at[pl.ds(h * RES_HALF, 4 * TOP_K)], out_hbm.at[img])


@functools.partial(
    pl.kernel,
    out_type=jax.ShapeDtypeStruct((B, 4 * TOP_K), jnp.float32),
    mesh=plsc.VectorSubcoreMesh(core_axis_name="c", subcore_axis_name="s"),
    compiler_params=pltpu.CompilerParams(needs_layout_passes=False),
    scratch_types=[
        pltpu.VMEM((16384,), jnp.int32),
        pltpu.VMEM((CAP_STRIP,), jnp.int32),
        pltpu.VMEM((CAP_STRIP,), jnp.int32),
        pltpu.VMEM((NBKT,), jnp.int32),
        pltpu.VMEM((NBKT,), jnp.int32),
        pltpu.VMEM((KCAP,), jnp.int32),
        pltpu.VMEM((KCAP,), jnp.int32),
        pltpu.VMEM((KCAP,), jnp.int32),
        pltpu.VMEM((KCAP,), jnp.int32),
        pltpu.VMEM((16,), jnp.int32),
        pltpu.VMEM((BAND_ELEMS,), jnp.float32),
        pltpu.VMEM((TOP_K,), jnp.int32),
        pltpu.VMEM((WCAP,), jnp.int32),
        pltpu.VMEM((WCAP,), jnp.int32),
        pltpu.VMEM((25, 16), jnp.float32),
        pltpu.VMEM((WCAP * 4,), jnp.float32),
        pltpu.VMEM((WCAP * 4,), jnp.int32),
        pltpu.VMEM((512,), jnp.int32),
        pltpu.VMEM_SHARED((2, 8, KEEP_CAP), jnp.int32),
        pltpu.VMEM_SHARED((2, 8, KEEP_CAP), jnp.int32),
        pltpu.VMEM_SHARED((2, 128), jnp.int32),
        pltpu.VMEM_SHARED((2, 8 * NBKT), jnp.int32),
        pltpu.VMEM_SHARED((2, TOP_K), jnp.int32),
        pltpu.VMEM_SHARED((2 * RES_HALF,), jnp.float32),
        pltpu.SemaphoreType.DMA,
    ],
)
def _sc_call(nms_hbm, img_hbm, out_hbm, *scratch):
  _sc_body(nms_hbm, img_hbm, out_hbm, *scratch)


@jax.jit
def kernel(scores_map):
  s3 = scores_map.reshape(B, H, W)
  nms = _nms_call(s3)
  out = _sc_call(nms.reshape(B * HW), s3.reshape(B * HW))
  o = out.reshape(B, TOP_K, 4)
  return o[:, :, 0:2], o[:, :, 2], o[:, :, 3]
